# Initial kernel scaffold; baseline (speedup 1.0000x reference)
import jax
import jax.numpy as jnp
from jax.experimental import pallas as pl


def kernel(nodes_feat, edge_index, edges_feat, nodes_num_norm_sqrt, edges_num_norm_sqrt, W_embed, b_embed, Wp, bp, Wa, ba, R0, rb0, R1, rb1, R2, rb2):
    raise NotImplementedError("write your pallas kernel here")


# speedup vs baseline: 5.2997x; 5.2997x over previous
import functools

import jax
import jax.numpy as jnp
from jax import lax
from jax.experimental import pallas as pl
from jax.experimental.pallas import tpu as pltpu
from jax.experimental.pallas import tpu_sc as plsc

NN = 50000
NE = 800000
HID = 64

CH = 128
KI = 8
TCH = 392
EPC = 16 * TCH
EP = EPC * CH
NPAD = 50048
DUMMY = 50000
TROW = 3128

_mesh = plsc.VectorSubcoreMesh(core_axis_name="c", subcore_axis_name="s")


def _zero_fill(ref, nrows, ncols):
    def row(j, _):
        for k in range(ncols // 16):
            ref[j, pl.ds(k * 16, 16)] = jnp.zeros((16,), jnp.float32)
        return 0
    lax.fori_loop(0, nrows, row, 0)


@functools.partial(
    pl.kernel,
    mesh=_mesh,
    out_type=jax.ShapeDtypeStruct((2, NPAD, 16), jnp.float32),
    scratch_types=[
        pltpu.VMEM((KI, CH), jnp.int32),
        pltpu.VMEM((CH, 16), jnp.float32),
        pltpu.VMEM((CH, 16), jnp.float32),
        pltpu.VMEM_SHARED((NPAD, 16), jnp.float32),
        pltpu.SemaphoreType.DMA,
    ],
    compiler_params=pltpu.CompilerParams(use_tc_tiling_on_sc=False),
)
def _sc_count(dstp, out, didx, ones_v, zb, cnt, sem):
    cid = lax.axis_index("c")
    sid = lax.axis_index("s")

    def fill(j, _):
        ones_v[j, pl.ds(0, 16)] = jnp.ones((16,), jnp.float32)
        zb[j, pl.ds(0, 16)] = jnp.zeros((16,), jnp.float32)
        return 0
    lax.fori_loop(0, CH, fill, 0)

    zbase = sid * TROW
    for t in range(24):
        pltpu.sync_copy(zb, cnt.at[pl.ds(zbase + t * CH, CH)])
    pltpu.sync_copy(zb.at[pl.ds(0, 56)], cnt.at[pl.ds(zbase + 24 * CH, 56)])
    plsc.subcore_barrier()

    def body(g, _):
        cb = cid * (EPC // 2) + sid * 192 + g * KI
        pltpu.sync_copy(dstp.at[pl.ds(cb, KI)], didx)
        for j in range(KI):
            pltpu.sync_copy(ones_v, cnt.at[didx.at[j]], add=True)
        return 0
    lax.fori_loop(0, 24, body, 0)

    @pl.when(sid < 8)
    def _():
        cb = cid * (EPC // 2) + 3072 + sid * KI
        pltpu.sync_copy(dstp.at[pl.ds(cb, KI)], didx)
        for j in range(KI):
            pltpu.sync_copy(ones_v, cnt.at[didx.at[j]], add=True)

    plsc.subcore_barrier()

    wb = sid * TROW
    pltpu.sync_copy(cnt.at[pl.ds(wb, TROW)], out.at[cid, pl.ds(wb, TROW)])


@functools.partial(
    pl.kernel,
    mesh=_mesh,
    out_type=jax.ShapeDtypeStruct((4, NPAD, 16), jnp.float32),
    scratch_types=[
        pltpu.VMEM((KI, CH), jnp.int32),
        pltpu.VMEM((KI, CH), jnp.int32),
        pltpu.VMEM((KI * CH, 16), jnp.float32),
        pltpu.VMEM((CH, 16), jnp.float32),
        pltpu.VMEM_SHARED((NPAD, 16), jnp.float32),
        pltpu.SemaphoreType.DMA,
    ],
    compiler_params=pltpu.CompilerParams(use_tc_tiling_on_sc=False),
)
def _sc_aggregate(q4, srcd, dstp, out, sidx, didx, rows, zb, acc, sem):
    cid = lax.axis_index("c")
    sid = lax.axis_index("s")

    _zero_fill(zb, CH, 16)
    for p in range(2):
        quarter = 2 * cid + p
        zbase = sid * TROW
        for t in range(24):
            pltpu.sync_copy(zb, acc.at[pl.ds(zbase + t * CH, CH)])
        pltpu.sync_copy(zb.at[pl.ds(0, 56)], acc.at[pl.ds(zbase + 24 * CH, 56)])
        plsc.subcore_barrier()

        def body(g, _):
            cb = sid * TCH + g * KI
            pltpu.sync_copy(srcd.at[quarter, pl.ds(cb, KI)], sidx)
            pltpu.sync_copy(dstp.at[pl.ds(cb, KI)], didx)
            cps = [
                pltpu.async_copy(q4.at[sidx.at[j]], rows.at[pl.ds(j * CH, CH)], sem)
                for j in range(KI)
            ]
            for cp in cps:
                cp.wait()
            for j in range(KI):
                pltpu.sync_copy(rows.at[pl.ds(j * CH, CH)],
                                acc.at[didx.at[j]], add=True)
            return 0
        lax.fori_loop(0, TCH // KI, body, 0)
        plsc.subcore_barrier()

        wb = sid * TROW
        pltpu.sync_copy(acc.at[pl.ds(wb, TROW)], out.at[quarter, pl.ds(wb, TROW)])


BN = 2000
GRID = NN // BN


def _full(shape):
    return pl.BlockSpec(shape, lambda i: tuple(0 for _ in shape))


def _split_q(q, q_out):
    for j in range(4):
        q_out[j] = q[:, 16 * j:16 * (j + 1)]


def _k0_body(nf, We, be, Wp0, bp0, h_out, q_out):
    h = jnp.dot(nf[...], We[...], preferred_element_type=jnp.float32) + be[...]
    q = jnp.maximum(
        jnp.dot(h, Wp0[...], preferred_element_type=jnp.float32) + bp0[...], 0.0)
    h_out[...] = h
    _split_q(q, q_out)


_k0 = pl.pallas_call(
    _k0_body,
    grid=(GRID,),
    in_specs=[
        pl.BlockSpec((BN, 128), lambda i: (i, 0)),
        _full((128, HID)),
        _full((1, HID)),
        _full((HID, HID)),
        _full((1, HID)),
    ],
    out_specs=[
        pl.BlockSpec((BN, HID), lambda i: (i, 0)),
        pl.BlockSpec((4, BN, 16), lambda i: (0, i, 0)),
    ],
    out_shape=[
        jax.ShapeDtypeStruct((NN, HID), jnp.float32),
        jax.ShapeDtypeStruct((4, NN, 16), jnp.float32),
    ],
)


def _node_apply(h, acc, cnt2, Wa, ba):
    cnt = cnt2[0, :, 0:1] + cnt2[1, :, 0:1]
    c = jnp.concatenate([acc[0], acc[1], acc[2], acc[3]], axis=1) \
        / jnp.maximum(cnt, 1.0)
    bundle = (
        jnp.dot(h, Wa[:HID, :], preferred_element_type=jnp.float32)
        + jnp.dot(c, Wa[HID:, :], preferred_element_type=jnp.float32)
        + ba
    )
    nrm = jnp.sqrt(jnp.sum(bundle * bundle, axis=1, keepdims=True))
    bundle = bundle / jnp.maximum(nrm, 1e-12)
    return h + jnp.maximum(bundle, 0.0)


def _kb_body(h_in, acc_in, cnt_in, Wa, ba, Wpn, bpn, h_out, q_out):
    hn = _node_apply(h_in[...], acc_in, cnt_in, Wa[...], ba[...])
    q = jnp.maximum(
        jnp.dot(hn, Wpn[...], preferred_element_type=jnp.float32) + bpn[...], 0.0)
    h_out[...] = hn
    _split_q(q, q_out)


_kb = pl.pallas_call(
    _kb_body,
    grid=(GRID,),
    in_specs=[
        pl.BlockSpec((BN, HID), lambda i: (i, 0)),
        pl.BlockSpec((4, BN, 16), lambda i: (0, i, 0)),
        pl.BlockSpec((2, BN, 16), lambda i: (0, i, 0)),
        _full((2 * HID, HID)),
        _full((1, HID)),
        _full((HID, HID)),
        _full((1, HID)),
    ],
    out_specs=[
        pl.BlockSpec((BN, HID), lambda i: (i, 0)),
        pl.BlockSpec((4, BN, 16), lambda i: (0, i, 0)),
    ],
    out_shape=[
        jax.ShapeDtypeStruct((NN, HID), jnp.float32),
        jax.ShapeDtypeStruct((4, NN, 16), jnp.float32),
    ],
)


def _kb3_body(h_in, acc_in, cnt_in, Wa, ba, h_out):
    h_out[...] = _node_apply(h_in[...], acc_in, cnt_in, Wa[...], ba[...])


_kb3 = pl.pallas_call(
    _kb3_body,
    grid=(GRID,),
    in_specs=[
        pl.BlockSpec((BN, HID), lambda i: (i, 0)),
        pl.BlockSpec((4, BN, 16), lambda i: (0, i, 0)),
        pl.BlockSpec((2, BN, 16), lambda i: (0, i, 0)),
        _full((2 * HID, HID)),
        _full((1, HID)),
    ],
    out_specs=pl.BlockSpec((BN, HID), lambda i: (i, 0)),
    out_shape=jax.ShapeDtypeStruct((NN, HID), jnp.float32),
)


def _kread_body(h, R0, rb0, R1, rb1, R2, rb2, out, accv):
    i = pl.program_id(0)

    @pl.when(i == 0)
    def _():
        accv[...] = jnp.zeros_like(accv)

    accv[0:1, :] = accv[0:1, :] + jnp.sum(h[...], axis=0, keepdims=True)

    @pl.when(i == GRID - 1)
    def _():
        hg = accv[0:1, :] * (1.0 / NN)
        y = jnp.maximum(
            jnp.dot(hg, R0[...], preferred_element_type=jnp.float32) + rb0[...], 0.0)
        y = jnp.maximum(
            jnp.dot(y, R1[...], preferred_element_type=jnp.float32) + rb1[...], 0.0)
        out[...] = jnp.dot(y, R2[...], preferred_element_type=jnp.float32) + rb2[...]


_kread = pl.pallas_call(
    _kread_body,
    grid=(GRID,),
    in_specs=[
        pl.BlockSpec((BN, HID), lambda i: (i, 0)),
        _full((HID, 32)),
        _full((1, 32)),
        _full((32, 16)),
        _full((1, 16)),
        _full((16, 10)),
        _full((1, 10)),
    ],
    out_specs=pl.BlockSpec((1, 10), lambda i: (0, 0)),
    out_shape=jax.ShapeDtypeStruct((1, 10), jnp.float32),
    scratch_shapes=[pltpu.VMEM((8, HID), jnp.float32)],
)


def kernel(nodes_feat, edge_index, edges_feat, nodes_num_norm_sqrt,
           edges_num_norm_sqrt, W_embed, b_embed, Wp, bp, Wa, ba,
           R0, rb0, R1, rb1, R2, rb2):
    src = edge_index[0]
    dst = edge_index[1]
    pad = EP - NE
    srcp = jnp.concatenate([src, jnp.zeros((pad,), jnp.int32)])
    dstp = jnp.concatenate([dst, jnp.full((pad,), DUMMY, jnp.int32)])
    srcd = jnp.stack([srcp + j * NN for j in range(4)]).reshape(4, EPC, CH)
    dstp3 = dstp.reshape(EPC, CH)

    cnt2 = _sc_count(dstp3)

    h, q2 = _k0(nodes_feat, W_embed, b_embed.reshape(1, HID),
                Wp[0], bp[0].reshape(1, HID))
    for l in range(4):
        acc2 = _sc_aggregate(q2.reshape(4 * NN, 16), srcd, dstp3)
        if l < 3:
            h, q2 = _kb(h, acc2, cnt2, Wa[l], ba[l].reshape(1, HID),
                        Wp[l + 1], bp[l + 1].reshape(1, HID))
        else:
            h = _kb3(h, acc2, cnt2, Wa[l], ba[l].reshape(1, HID))

    return _kread(h, R0, rb0.reshape(1, 32), R1, rb1.reshape(1, 16),
                  R2, rb2.reshape(1, 10))


# speedup vs baseline: 6.7670x; 1.2769x over previous
import functools

import jax
import jax.numpy as jnp
from jax import lax
from jax.experimental import pallas as pl
from jax.experimental.pallas import tpu as pltpu
from jax.experimental.pallas import tpu_sc as plsc

NN = 50000
NE = 800000
HID = 64

CH = 128
KI = 7
NBLK = 56
NPAIR = NBLK // 2
TCH = 392
EPC = 16 * TCH
EP = EPC * CH
NPAD = 50048
DUMMY = 50000
TROW = 3128

_mesh = plsc.VectorSubcoreMesh(core_axis_name="c", subcore_axis_name="s")


def _zero_fill(ref, nrows, ncols):
    def row(j, _):
        for k in range(ncols // 16):
            ref[j, pl.ds(k * 16, 16)] = jnp.zeros((16,), jnp.float32)
        return 0
    lax.fori_loop(0, nrows, row, 0)


@functools.partial(
    pl.kernel,
    mesh=_mesh,
    out_type=jax.ShapeDtypeStruct((2, NPAD, 16), jnp.float32),
    scratch_types=[
        pltpu.VMEM((8, CH), jnp.int32),
        pltpu.VMEM((CH, 16), jnp.float32),
        pltpu.VMEM((CH, 16), jnp.float32),
        pltpu.VMEM_SHARED((NPAD, 16), jnp.float32),
        pltpu.SemaphoreType.DMA,
    ],
    compiler_params=pltpu.CompilerParams(use_tc_tiling_on_sc=False),
)
def _sc_count(dstp, out, didx, ones_v, zb, cnt, sem):
    cid = lax.axis_index("c")
    sid = lax.axis_index("s")

    def fill(j, _):
        ones_v[j, pl.ds(0, 16)] = jnp.ones((16,), jnp.float32)
        zb[j, pl.ds(0, 16)] = jnp.zeros((16,), jnp.float32)
        return 0
    lax.fori_loop(0, CH, fill, 0)

    zbase = sid * TROW
    for t in range(24):
        pltpu.sync_copy(zb, cnt.at[pl.ds(zbase + t * CH, CH)])
    pltpu.sync_copy(zb.at[pl.ds(0, 56)], cnt.at[pl.ds(zbase + 24 * CH, 56)])
    plsc.subcore_barrier()

    def body(g, _):
        cb = cid * (EPC // 2) + sid * 192 + g * 8
        pltpu.sync_copy(dstp.at[pl.ds(cb, 8)], didx)
        for j in range(8):
            pltpu.sync_copy(ones_v, cnt.at[didx.at[j]], add=True)
        return 0
    lax.fori_loop(0, 24, body, 0)

    @pl.when(sid < 8)
    def _():
        cb = cid * (EPC // 2) + 3072 + sid * 8
        pltpu.sync_copy(dstp.at[pl.ds(cb, 8)], didx)
        for j in range(8):
            pltpu.sync_copy(ones_v, cnt.at[didx.at[j]], add=True)

    plsc.subcore_barrier()

    wb = sid * TROW
    pltpu.sync_copy(cnt.at[pl.ds(wb, TROW)], out.at[cid, pl.ds(wb, TROW)])


@functools.partial(
    pl.kernel,
    mesh=_mesh,
    out_type=jax.ShapeDtypeStruct((4, NPAD, 16), jnp.float32),
    scratch_types=[
        pltpu.VMEM((2, KI, CH), jnp.int32),
        pltpu.VMEM((2, KI, CH), jnp.int32),
        pltpu.VMEM((2, KI * CH, 16), jnp.float32),
        pltpu.VMEM((CH, 16), jnp.float32),
        pltpu.VMEM_SHARED((NPAD, 16), jnp.float32),
        pltpu.SemaphoreType.DMA,
        pltpu.SemaphoreType.DMA,
    ],
    compiler_params=pltpu.CompilerParams(use_tc_tiling_on_sc=False),
)
def _sc_aggregate(q4, srcd, dstp, out, sidx, didx, rows, zb, acc, semA, semB):
    cid = lax.axis_index("c")
    sid = lax.axis_index("s")

    _zero_fill(zb, CH, 16)
    for p in range(2):
        quarter = 2 * cid + p
        zbase = sid * TROW
        for t in range(24):
            pltpu.sync_copy(zb, acc.at[pl.ds(zbase + t * CH, CH)])
        pltpu.sync_copy(zb.at[pl.ds(0, 56)], acc.at[pl.ds(zbase + 24 * CH, 56)])
        plsc.subcore_barrier()

        base = sid * TCH

        def load_and_fire(buf, cb, sem):
            pltpu.sync_copy(srcd.at[quarter, pl.ds(cb, KI)], sidx.at[buf])
            pltpu.sync_copy(dstp.at[pl.ds(cb, KI)], didx.at[buf])
            for j in range(KI):
                pltpu.async_copy(q4.at[sidx.at[buf, j]],
                                 rows.at[buf, pl.ds(j * CH, CH)], sem)

        def drain(buf, sem):
            for j in range(KI):
                pltpu.make_async_copy(q4.at[sidx.at[buf, j]],
                                      rows.at[buf, pl.ds(j * CH, CH)], sem).wait()

        def scatter(buf):
            for j in range(KI):
                pltpu.sync_copy(rows.at[buf, pl.ds(j * CH, CH)],
                                acc.at[didx.at[buf, j]], add=True)

        load_and_fire(0, base, semA)

        def body(b, _):
            load_and_fire(1, base + (2 * b + 1) * KI, semB)
            drain(0, semA)
            scatter(0)

            @pl.when(b < NPAIR - 1)
            def _():
                load_and_fire(0, base + (2 * b + 2) * KI, semA)

            drain(1, semB)
            scatter(1)
            return 0
        lax.fori_loop(0, NPAIR, body, 0)
        plsc.subcore_barrier()

        wb = sid * TROW
        pltpu.sync_copy(acc.at[pl.ds(wb, TROW)], out.at[quarter, pl.ds(wb, TROW)])


BN = 2000
GRID = NN // BN


def _full(shape):
    return pl.BlockSpec(shape, lambda i: tuple(0 for _ in shape))


def _split_q(q, q_out):
    for j in range(4):
        q_out[j] = q[:, 16 * j:16 * (j + 1)]


def _k0_body(nf, We, be, Wp0, bp0, h_out, q_out):
    h = jnp.dot(nf[...], We[...], preferred_element_type=jnp.float32) + be[...]
    q = jnp.maximum(
        jnp.dot(h, Wp0[...], preferred_element_type=jnp.float32) + bp0[...], 0.0)
    h_out[...] = h
    _split_q(q, q_out)


_k0 = pl.pallas_call(
    _k0_body,
    grid=(GRID,),
    in_specs=[
        pl.BlockSpec((BN, 128), lambda i: (i, 0)),
        _full((128, HID)),
        _full((1, HID)),
        _full((HID, HID)),
        _full((1, HID)),
    ],
    out_specs=[
        pl.BlockSpec((BN, HID), lambda i: (i, 0)),
        pl.BlockSpec((4, BN, 16), lambda i: (0, i, 0)),
    ],
    out_shape=[
        jax.ShapeDtypeStruct((NN, HID), jnp.float32),
        jax.ShapeDtypeStruct((4, NN, 16), jnp.float32),
    ],
)


def _node_apply(h, acc, cnt2, Wa, ba):
    cnt = cnt2[0, :, 0:1] + cnt2[1, :, 0:1]
    c = jnp.concatenate([acc[0], acc[1], acc[2], acc[3]], axis=1) \
        / jnp.maximum(cnt, 1.0)
    bundle = (
        jnp.dot(h, Wa[:HID, :], preferred_element_type=jnp.float32)
        + jnp.dot(c, Wa[HID:, :], preferred_element_type=jnp.float32)
        + ba
    )
    nrm = jnp.sqrt(jnp.sum(bundle * bundle, axis=1, keepdims=True))
    bundle = bundle / jnp.maximum(nrm, 1e-12)
    return h + jnp.maximum(bundle, 0.0)


def _kb_body(h_in, acc_in, cnt_in, Wa, ba, Wpn, bpn, h_out, q_out):
    hn = _node_apply(h_in[...], acc_in, cnt_in, Wa[...], ba[...])
    q = jnp.maximum(
        jnp.dot(hn, Wpn[...], preferred_element_type=jnp.float32) + bpn[...], 0.0)
    h_out[...] = hn
    _split_q(q, q_out)


_kb = pl.pallas_call(
    _kb_body,
    grid=(GRID,),
    in_specs=[
        pl.BlockSpec((BN, HID), lambda i: (i, 0)),
        pl.BlockSpec((4, BN, 16), lambda i: (0, i, 0)),
        pl.BlockSpec((2, BN, 16), lambda i: (0, i, 0)),
        _full((2 * HID, HID)),
        _full((1, HID)),
        _full((HID, HID)),
        _full((1, HID)),
    ],
    out_specs=[
        pl.BlockSpec((BN, HID), lambda i: (i, 0)),
        pl.BlockSpec((4, BN, 16), lambda i: (0, i, 0)),
    ],
    out_shape=[
        jax.ShapeDtypeStruct((NN, HID), jnp.float32),
        jax.ShapeDtypeStruct((4, NN, 16), jnp.float32),
    ],
)


def _kb3_body(h_in, acc_in, cnt_in, Wa, ba, h_out):
    h_out[...] = _node_apply(h_in[...], acc_in, cnt_in, Wa[...], ba[...])


_kb3 = pl.pallas_call(
    _kb3_body,
    grid=(GRID,),
    in_specs=[
        pl.BlockSpec((BN, HID), lambda i: (i, 0)),
        pl.BlockSpec((4, BN, 16), lambda i: (0, i, 0)),
        pl.BlockSpec((2, BN, 16), lambda i: (0, i, 0)),
        _full((2 * HID, HID)),
        _full((1, HID)),
    ],
    out_specs=pl.BlockSpec((BN, HID), lambda i: (i, 0)),
    out_shape=jax.ShapeDtypeStruct((NN, HID), jnp.float32),
)


def _kread_body(h, R0, rb0, R1, rb1, R2, rb2, out, accv):
    i = pl.program_id(0)

    @pl.when(i == 0)
    def _():
        accv[...] = jnp.zeros_like(accv)

    accv[0:1, :] = accv[0:1, :] + jnp.sum(h[...], axis=0, keepdims=True)

    @pl.when(i == GRID - 1)
    def _():
        hg = accv[0:1, :] * (1.0 / NN)
        y = jnp.maximum(
            jnp.dot(hg, R0[...], preferred_element_type=jnp.float32) + rb0[...], 0.0)
        y = jnp.maximum(
            jnp.dot(y, R1[...], preferred_element_type=jnp.float32) + rb1[...], 0.0)
        out[...] = jnp.dot(y, R2[...], preferred_element_type=jnp.float32) + rb2[...]


_kread = pl.pallas_call(
    _kread_body,
    grid=(GRID,),
    in_specs=[
        pl.BlockSpec((BN, HID), lambda i: (i, 0)),
        _full((HID, 32)),
        _full((1, 32)),
        _full((32, 16)),
        _full((1, 16)),
        _full((16, 10)),
        _full((1, 10)),
    ],
    out_specs=pl.BlockSpec((1, 10), lambda i: (0, 0)),
    out_shape=jax.ShapeDtypeStruct((1, 10), jnp.float32),
    scratch_shapes=[pltpu.VMEM((8, HID), jnp.float32)],
)


def kernel(nodes_feat, edge_index, edges_feat, nodes_num_norm_sqrt,
           edges_num_norm_sqrt, W_embed, b_embed, Wp, bp, Wa, ba,
           R0, rb0, R1, rb1, R2, rb2):
    src = edge_index[0]
    dst = edge_index[1]
    pad = EP - NE
    srcp = jnp.concatenate([src, jnp.zeros((pad,), jnp.int32)])
    dstp = jnp.concatenate([dst, jnp.full((pad,), DUMMY, jnp.int32)])
    srcd = jnp.stack([srcp + j * NN for j in range(4)]).reshape(4, EPC, CH)
    dstp3 = dstp.reshape(EPC, CH)

    cnt2 = _sc_count(dstp3)

    h, q2 = _k0(nodes_feat, W_embed, b_embed.reshape(1, HID),
                Wp[0], bp[0].reshape(1, HID))
    for l in range(4):
        acc2 = _sc_aggregate(q2.reshape(4 * NN, 16), srcd, dstp3)
        if l < 3:
            h, q2 = _kb(h, acc2, cnt2, Wa[l], ba[l].reshape(1, HID),
                        Wp[l + 1], bp[l + 1].reshape(1, HID))
        else:
            h = _kb3(h, acc2, cnt2, Wa[l], ba[l].reshape(1, HID))

    return _kread(h, R0, rb0.reshape(1, 32), R1, rb1.reshape(1, 16),
                  R2, rb2.reshape(1, 10))


# speedup vs baseline: 8.5092x; 1.2575x over previous
import functools

import jax
import jax.numpy as jnp
from jax import lax
from jax.experimental import pallas as pl
from jax.experimental.pallas import tpu as pltpu
from jax.experimental.pallas import tpu_sc as plsc

NN = 50000
NE = 800000
HID = 64

CH = 128
KI = 7
NBLK = 56
NPAIR = NBLK // 2
TCH = 392
EPC = 16 * TCH
EP = EPC * CH
NPAD = 50048
DUMMY = 50000
TROW = 3128

_mesh = plsc.VectorSubcoreMesh(core_axis_name="c", subcore_axis_name="s")


def _zero_fill(ref, nrows, ncols):
    def row(j, _):
        for k in range(ncols // 16):
            ref[j, pl.ds(k * 16, 16)] = jnp.zeros((16,), jnp.float32)
        return 0
    lax.fori_loop(0, nrows, row, 0)


@functools.partial(
    pl.kernel,
    mesh=_mesh,
    out_type=jax.ShapeDtypeStruct((NPAD, 128), jnp.float32),
    scratch_types=[
        pltpu.VMEM((8, CH), jnp.int32),
        pltpu.VMEM((CH, 16), jnp.float32),
        pltpu.VMEM((CH, 16), jnp.float32),
        pltpu.VMEM_SHARED((NPAD, 16), jnp.float32),
        pltpu.SemaphoreType.DMA,
    ],
    compiler_params=pltpu.CompilerParams(use_tc_tiling_on_sc=False),
)
def _sc_count(dstp, out, didx, ones_v, zb, cnt, sem):
    cid = lax.axis_index("c")
    sid = lax.axis_index("s")

    def fill(j, _):
        ones_v[j, pl.ds(0, 16)] = jnp.ones((16,), jnp.float32)
        zb[j, pl.ds(0, 16)] = jnp.zeros((16,), jnp.float32)
        return 0
    lax.fori_loop(0, CH, fill, 0)

    zbase = sid * TROW
    for t in range(24):
        pltpu.sync_copy(zb, cnt.at[pl.ds(zbase + t * CH, CH)])
    pltpu.sync_copy(zb.at[pl.ds(0, 56)], cnt.at[pl.ds(zbase + 24 * CH, 56)])
    plsc.subcore_barrier()

    def body(g, _):
        cb = cid * (EPC // 2) + sid * 192 + g * 8
        pltpu.sync_copy(dstp.at[pl.ds(cb, 8)], didx)
        for j in range(8):
            pltpu.sync_copy(ones_v, cnt.at[didx.at[j]], add=True)
        return 0
    lax.fori_loop(0, 24, body, 0)

    @pl.when(sid < 8)
    def _():
        cb = cid * (EPC // 2) + 3072 + sid * 8
        pltpu.sync_copy(dstp.at[pl.ds(cb, 8)], didx)
        for j in range(8):
            pltpu.sync_copy(ones_v, cnt.at[didx.at[j]], add=True)

    plsc.subcore_barrier()

    wb = sid * TROW
    pltpu.sync_copy(cnt.at[pl.ds(wb, TROW)],
                    out.at[pl.ds(wb, TROW), pl.ds(16 * cid, 16)])


@functools.partial(
    pl.kernel,
    mesh=_mesh,
    out_type=jax.ShapeDtypeStruct((NPAD, 128), jnp.float32),
    scratch_types=[
        pltpu.VMEM((2, KI, CH), jnp.int32),
        pltpu.VMEM((2, KI, CH), jnp.int32),
        pltpu.VMEM((2, KI * CH, 16), jnp.float32),
        pltpu.VMEM((CH, 16), jnp.float32),
        pltpu.VMEM_SHARED((NPAD, 16), jnp.float32),
        pltpu.SemaphoreType.DMA,
        pltpu.SemaphoreType.DMA,
    ],
    compiler_params=pltpu.CompilerParams(use_tc_tiling_on_sc=False),
)
def _sc_aggregate(q4, srcd, dstp, out, sidx, didx, rows, zb, acc, semA, semB):
    cid = lax.axis_index("c")
    sid = lax.axis_index("s")

    _zero_fill(zb, CH, 16)
    for p in range(2):
        quarter = 2 * cid + p
        zbase = sid * TROW
        for t in range(24):
            pltpu.sync_copy(zb, acc.at[pl.ds(zbase + t * CH, CH)])
        pltpu.sync_copy(zb.at[pl.ds(0, 56)], acc.at[pl.ds(zbase + 24 * CH, 56)])
        plsc.subcore_barrier()

        base = sid * TCH

        def load_and_fire(buf, cb, sem):
            pltpu.sync_copy(srcd.at[quarter, pl.ds(cb, KI)], sidx.at[buf])
            pltpu.sync_copy(dstp.at[pl.ds(cb, KI)], didx.at[buf])
            for j in range(KI):
                pltpu.async_copy(q4.at[sidx.at[buf, j]],
                                 rows.at[buf, pl.ds(j * CH, CH)], sem)

        def drain(buf, sem):
            for j in range(KI):
                pltpu.make_async_copy(q4.at[sidx.at[buf, j]],
                                      rows.at[buf, pl.ds(j * CH, CH)], sem).wait()

        def scatter(buf):
            for j in range(KI):
                pltpu.sync_copy(rows.at[buf, pl.ds(j * CH, CH)],
                                acc.at[didx.at[buf, j]], add=True)

        load_and_fire(0, base, semA)

        def body(b, _):
            load_and_fire(1, base + (2 * b + 1) * KI, semB)
            drain(0, semA)
            scatter(0)

            @pl.when(b < NPAIR - 1)
            def _():
                load_and_fire(0, base + (2 * b + 2) * KI, semA)

            drain(1, semB)
            scatter(1)
            return 0
        lax.fori_loop(0, NPAIR, body, 0)
        plsc.subcore_barrier()

        wb = sid * TROW
        pltpu.sync_copy(acc.at[pl.ds(wb, TROW)],
                        out.at[pl.ds(wb, TROW), pl.ds(16 * quarter, 16)])


BN = 2000
GRID = NN // BN


def _full(shape):
    return pl.BlockSpec(shape, lambda i: tuple(0 for _ in shape))


def _split_q(q, q_out):
    q_out[...] = jnp.concatenate([q, q], axis=1)


def _k0_body(nf, We, be, Wp0, bp0, h_out, q_out):
    h = jnp.dot(nf[...], We[...], preferred_element_type=jnp.float32) + be[...]
    q = jnp.maximum(
        jnp.dot(h, Wp0[...], preferred_element_type=jnp.float32) + bp0[...], 0.0)
    h_out[...] = h
    _split_q(q, q_out)


_k0 = pl.pallas_call(
    _k0_body,
    grid=(GRID,),
    in_specs=[
        pl.BlockSpec((BN, 128), lambda i: (i, 0)),
        _full((128, HID)),
        _full((1, HID)),
        _full((HID, HID)),
        _full((1, HID)),
    ],
    out_specs=[
        pl.BlockSpec((BN, HID), lambda i: (i, 0)),
        pl.BlockSpec((BN, 128), lambda i: (i, 0)),
    ],
    out_shape=[
        jax.ShapeDtypeStruct((NN, HID), jnp.float32),
        jax.ShapeDtypeStruct((NN, 128), jnp.float32),
    ],
)


def _node_apply(h, acc, cnt2, Wa, ba):
    cnt = cnt2[:, 0:1] + cnt2[:, 16:17]
    c = acc[:, 0:HID] / jnp.maximum(cnt, 1.0)
    bundle = (
        jnp.dot(h, Wa[:HID, :], preferred_element_type=jnp.float32)
        + jnp.dot(c, Wa[HID:, :], preferred_element_type=jnp.float32)
        + ba
    )
    nrm = jnp.sqrt(jnp.sum(bundle * bundle, axis=1, keepdims=True))
    bundle = bundle / jnp.maximum(nrm, 1e-12)
    return h + jnp.maximum(bundle, 0.0)


def _kb_body(h_in, acc_in, cnt_in, Wa, ba, Wpn, bpn, h_out, q_out):
    hn = _node_apply(h_in[...], acc_in[...], cnt_in[...], Wa[...], ba[...])
    q = jnp.maximum(
        jnp.dot(hn, Wpn[...], preferred_element_type=jnp.float32) + bpn[...], 0.0)
    h_out[...] = hn
    _split_q(q, q_out)


_kb = pl.pallas_call(
    _kb_body,
    grid=(GRID,),
    in_specs=[
        pl.BlockSpec((BN, HID), lambda i: (i, 0)),
        pl.BlockSpec((BN, 128), lambda i: (i, 0)),
        pl.BlockSpec((BN, 128), lambda i: (i, 0)),
        _full((2 * HID, HID)),
        _full((1, HID)),
        _full((HID, HID)),
        _full((1, HID)),
    ],
    out_specs=[
        pl.BlockSpec((BN, HID), lambda i: (i, 0)),
        pl.BlockSpec((BN, 128), lambda i: (i, 0)),
    ],
    out_shape=[
        jax.ShapeDtypeStruct((NN, HID), jnp.float32),
        jax.ShapeDtypeStruct((NN, 128), jnp.float32),
    ],
)


def _kb3_body(h_in, acc_in, cnt_in, Wa, ba, h_out):
    h_out[...] = _node_apply(h_in[...], acc_in[...], cnt_in[...], Wa[...], ba[...])


_kb3 = pl.pallas_call(
    _kb3_body,
    grid=(GRID,),
    in_specs=[
        pl.BlockSpec((BN, HID), lambda i: (i, 0)),
        pl.BlockSpec((BN, 128), lambda i: (i, 0)),
        pl.BlockSpec((BN, 128), lambda i: (i, 0)),
        _full((2 * HID, HID)),
        _full((1, HID)),
    ],
    out_specs=pl.BlockSpec((BN, HID), lambda i: (i, 0)),
    out_shape=jax.ShapeDtypeStruct((NN, HID), jnp.float32),
)


def _kread_body(h, R0, rb0, R1, rb1, R2, rb2, out, accv):
    i = pl.program_id(0)

    @pl.when(i == 0)
    def _():
        accv[...] = jnp.zeros_like(accv)

    accv[0:1, :] = accv[0:1, :] + jnp.sum(h[...], axis=0, keepdims=True)

    @pl.when(i == GRID - 1)
    def _():
        hg = accv[0:1, :] * (1.0 / NN)
        y = jnp.maximum(
            jnp.dot(hg, R0[...], preferred_element_type=jnp.float32) + rb0[...], 0.0)
        y = jnp.maximum(
            jnp.dot(y, R1[...], preferred_element_type=jnp.float32) + rb1[...], 0.0)
        out[...] = jnp.dot(y, R2[...], preferred_element_type=jnp.float32) + rb2[...]


_kread = pl.pallas_call(
    _kread_body,
    grid=(GRID,),
    in_specs=[
        pl.BlockSpec((BN, HID), lambda i: (i, 0)),
        _full((HID, 32)),
        _full((1, 32)),
        _full((32, 16)),
        _full((1, 16)),
        _full((16, 10)),
        _full((1, 10)),
    ],
    out_specs=pl.BlockSpec((1, 10), lambda i: (0, 0)),
    out_shape=jax.ShapeDtypeStruct((1, 10), jnp.float32),
    scratch_shapes=[pltpu.VMEM((8, HID), jnp.float32)],
)


def kernel(nodes_feat, edge_index, edges_feat, nodes_num_norm_sqrt,
           edges_num_norm_sqrt, W_embed, b_embed, Wp, bp, Wa, ba,
           R0, rb0, R1, rb1, R2, rb2):
    src = edge_index[0]
    dst = edge_index[1]
    pad = EP - NE
    srcp = jnp.concatenate([src, jnp.zeros((pad,), jnp.int32)])
    dstp = jnp.concatenate([dst, jnp.full((pad,), DUMMY, jnp.int32)])
    srcd = jnp.stack([srcp * 8 + j for j in range(4)]).reshape(4, EPC, CH)
    dstp3 = dstp.reshape(EPC, CH)

    cnt2 = _sc_count(dstp3)

    h, q2 = _k0(nodes_feat, W_embed, b_embed.reshape(1, HID),
                Wp[0], bp[0].reshape(1, HID))
    for l in range(4):
        acc2 = _sc_aggregate(q2.reshape(8 * NN, 16), srcd, dstp3)
        if l < 3:
            h, q2 = _kb(h, acc2, cnt2, Wa[l], ba[l].reshape(1, HID),
                        Wp[l + 1], bp[l + 1].reshape(1, HID))
        else:
            h = _kb3(h, acc2, cnt2, Wa[l], ba[l].reshape(1, HID))

    return _kread(h, R0, rb0.reshape(1, 32), R1, rb1.reshape(1, 16),
                  R2, rb2.reshape(1, 10))


# speedup vs baseline: 9.1928x; 1.0803x over previous
import functools

import jax
import jax.numpy as jnp
from jax import lax
from jax.experimental import pallas as pl
from jax.experimental.pallas import tpu as pltpu
from jax.experimental.pallas import tpu_sc as plsc

NN = 50000
NE = 800000
HID = 64

CH = 128
KI = 7
BLK = KI * CH
NBLK = 56
NPAIR = NBLK // 2
TCH = 392
EPC = 16 * TCH
EP = EPC * CH
NPAD = 50048
DUMMY = 50000
TROW = 3128

_mesh = plsc.VectorSubcoreMesh(core_axis_name="c", subcore_axis_name="s")


def _zero_fill(ref, nrows, ncols):
    def row(j, _):
        for k in range(ncols // 16):
            ref[j, pl.ds(k * 16, 16)] = jnp.zeros((16,), jnp.float32)
        return 0
    lax.fori_loop(0, nrows, row, 0)


@functools.partial(
    pl.kernel,
    mesh=_mesh,
    out_type=jax.ShapeDtypeStruct((NPAD, 128), jnp.float32),
    scratch_types=[
        pltpu.VMEM((8, CH), jnp.int32),
        pltpu.VMEM((CH, 16), jnp.float32),
        pltpu.VMEM((CH, 16), jnp.float32),
        pltpu.VMEM_SHARED((NPAD, 16), jnp.float32),
        pltpu.SemaphoreType.DMA,
    ],
    compiler_params=pltpu.CompilerParams(use_tc_tiling_on_sc=False),
)
def _sc_count(dstp, out, didx, ones_v, zb, cnt, sem):
    cid = lax.axis_index("c")
    sid = lax.axis_index("s")

    def fill(j, _):
        ones_v[j, pl.ds(0, 16)] = jnp.ones((16,), jnp.float32)
        zb[j, pl.ds(0, 16)] = jnp.zeros((16,), jnp.float32)
        return 0
    lax.fori_loop(0, CH, fill, 0)

    zbase = sid * TROW
    for t in range(24):
        pltpu.sync_copy(zb, cnt.at[pl.ds(zbase + t * CH, CH)])
    pltpu.sync_copy(zb.at[pl.ds(0, 56)], cnt.at[pl.ds(zbase + 24 * CH, 56)])
    plsc.subcore_barrier()

    def body(g, _):
        cb = cid * (EPC // 2) + sid * 192 + g * 8
        pltpu.sync_copy(dstp.at[pl.ds(cb, 8)], didx)
        for j in range(8):
            pltpu.sync_copy(ones_v, cnt.at[didx.at[j]], add=True)
        return 0
    lax.fori_loop(0, 24, body, 0)

    @pl.when(sid < 8)
    def _():
        cb = cid * (EPC // 2) + 3072 + sid * 8
        pltpu.sync_copy(dstp.at[pl.ds(cb, 8)], didx)
        for j in range(8):
            pltpu.sync_copy(ones_v, cnt.at[didx.at[j]], add=True)

    plsc.subcore_barrier()

    wb = sid * TROW
    pltpu.sync_copy(cnt.at[pl.ds(wb, TROW)],
                    out.at[pl.ds(wb, TROW), pl.ds(16 * cid, 16)])


@functools.partial(
    pl.kernel,
    mesh=_mesh,
    out_type=jax.ShapeDtypeStruct((NPAD, 128), jnp.float32),
    scratch_types=[
        pltpu.VMEM((2, BLK), jnp.int32),
        pltpu.VMEM((2, BLK), jnp.int32),
        pltpu.VMEM((2, BLK, 16), jnp.float32),
        pltpu.VMEM((CH, 16), jnp.float32),
        pltpu.VMEM_SHARED((NPAD, 16), jnp.float32),
        pltpu.SemaphoreType.DMA,
        pltpu.SemaphoreType.DMA,
    ],
    compiler_params=pltpu.CompilerParams(use_tc_tiling_on_sc=False),
)
def _sc_aggregate(q4, srcd, dstp, out, sidx, didx, rows, zb, acc, semA, semB):
    cid = lax.axis_index("c")
    sid = lax.axis_index("s")

    _zero_fill(zb, CH, 16)
    for p in range(2):
        quarter = 2 * cid + p
        zbase = sid * TROW
        for t in range(24):
            pltpu.sync_copy(zb, acc.at[pl.ds(zbase + t * CH, CH)])
        pltpu.sync_copy(zb.at[pl.ds(0, 56)], acc.at[pl.ds(zbase + 24 * CH, 56)])
        plsc.subcore_barrier()

        base = sid * NBLK

        def load_and_fire(buf, cb, sem):
            pltpu.sync_copy(srcd.at[quarter, cb], sidx.at[buf])
            pltpu.sync_copy(dstp.at[cb], didx.at[buf])
            pltpu.async_copy(q4.at[sidx.at[buf]], rows.at[buf], sem)

        def drain(buf, sem):
            pltpu.make_async_copy(q4.at[sidx.at[buf]], rows.at[buf], sem).wait()

        def scatter(buf):
            pltpu.sync_copy(rows.at[buf], acc.at[didx.at[buf]], add=True)

        load_and_fire(0, base, semA)

        def body(b, _):
            load_and_fire(1, base + (2 * b + 1), semB)
            drain(0, semA)
            scatter(0)

            @pl.when(b < NPAIR - 1)
            def _():
                load_and_fire(0, base + (2 * b + 2), semA)

            drain(1, semB)
            scatter(1)
            return 0
        lax.fori_loop(0, NPAIR, body, 0)
        plsc.subcore_barrier()

        wb = sid * TROW
        pltpu.sync_copy(acc.at[pl.ds(wb, TROW)],
                        out.at[pl.ds(wb, TROW), pl.ds(16 * quarter, 16)])


BN = 2000
GRID = NN // BN


def _full(shape):
    return pl.BlockSpec(shape, lambda i: tuple(0 for _ in shape))


def _split_q(q, q_out):
    q_out[...] = jnp.concatenate([q, q], axis=1)


def _k0_body(nf, We, be, Wp0, bp0, h_out, q_out):
    h = jnp.dot(nf[...], We[...], preferred_element_type=jnp.float32) + be[...]
    q = jnp.maximum(
        jnp.dot(h, Wp0[...], preferred_element_type=jnp.float32) + bp0[...], 0.0)
    h_out[...] = h
    _split_q(q, q_out)


_k0 = pl.pallas_call(
    _k0_body,
    grid=(GRID,),
    in_specs=[
        pl.BlockSpec((BN, 128), lambda i: (i, 0)),
        _full((128, HID)),
        _full((1, HID)),
        _full((HID, HID)),
        _full((1, HID)),
    ],
    out_specs=[
        pl.BlockSpec((BN, HID), lambda i: (i, 0)),
        pl.BlockSpec((BN, 128), lambda i: (i, 0)),
    ],
    out_shape=[
        jax.ShapeDtypeStruct((NN, HID), jnp.float32),
        jax.ShapeDtypeStruct((NN, 128), jnp.float32),
    ],
)


def _node_apply(h, acc, cnt2, Wa, ba):
    cnt = cnt2[:, 0:1] + cnt2[:, 16:17]
    c = acc[:, 0:HID] / jnp.maximum(cnt, 1.0)
    bundle = (
        jnp.dot(h, Wa[:HID, :], preferred_element_type=jnp.float32)
        + jnp.dot(c, Wa[HID:, :], preferred_element_type=jnp.float32)
        + ba
    )
    nrm = jnp.sqrt(jnp.sum(bundle * bundle, axis=1, keepdims=True))
    bundle = bundle / jnp.maximum(nrm, 1e-12)
    return h + jnp.maximum(bundle, 0.0)


def _kb_body(h_in, acc_in, cnt_in, Wa, ba, Wpn, bpn, h_out, q_out):
    hn = _node_apply(h_in[...], acc_in[...], cnt_in[...], Wa[...], ba[...])
    q = jnp.maximum(
        jnp.dot(hn, Wpn[...], preferred_element_type=jnp.float32) + bpn[...], 0.0)
    h_out[...] = hn
    _split_q(q, q_out)


_kb = pl.pallas_call(
    _kb_body,
    grid=(GRID,),
    in_specs=[
        pl.BlockSpec((BN, HID), lambda i: (i, 0)),
        pl.BlockSpec((BN, 128), lambda i: (i, 0)),
        pl.BlockSpec((BN, 128), lambda i: (i, 0)),
        _full((2 * HID, HID)),
        _full((1, HID)),
        _full((HID, HID)),
        _full((1, HID)),
    ],
    out_specs=[
        pl.BlockSpec((BN, HID), lambda i: (i, 0)),
        pl.BlockSpec((BN, 128), lambda i: (i, 0)),
    ],
    out_shape=[
        jax.ShapeDtypeStruct((NN, HID), jnp.float32),
        jax.ShapeDtypeStruct((NN, 128), jnp.float32),
    ],
)


def _kb3_body(h_in, acc_in, cnt_in, Wa, ba, h_out):
    h_out[...] = _node_apply(h_in[...], acc_in[...], cnt_in[...], Wa[...], ba[...])


_kb3 = pl.pallas_call(
    _kb3_body,
    grid=(GRID,),
    in_specs=[
        pl.BlockSpec((BN, HID), lambda i: (i, 0)),
        pl.BlockSpec((BN, 128), lambda i: (i, 0)),
        pl.BlockSpec((BN, 128), lambda i: (i, 0)),
        _full((2 * HID, HID)),
        _full((1, HID)),
    ],
    out_specs=pl.BlockSpec((BN, HID), lambda i: (i, 0)),
    out_shape=jax.ShapeDtypeStruct((NN, HID), jnp.float32),
)


def _kread_body(h, R0, rb0, R1, rb1, R2, rb2, out, accv):
    i = pl.program_id(0)

    @pl.when(i == 0)
    def _():
        accv[...] = jnp.zeros_like(accv)

    accv[0:1, :] = accv[0:1, :] + jnp.sum(h[...], axis=0, keepdims=True)

    @pl.when(i == GRID - 1)
    def _():
        hg = accv[0:1, :] * (1.0 / NN)
        y = jnp.maximum(
            jnp.dot(hg, R0[...], preferred_element_type=jnp.float32) + rb0[...], 0.0)
        y = jnp.maximum(
            jnp.dot(y, R1[...], preferred_element_type=jnp.float32) + rb1[...], 0.0)
        out[...] = jnp.dot(y, R2[...], preferred_element_type=jnp.float32) + rb2[...]


_kread = pl.pallas_call(
    _kread_body,
    grid=(GRID,),
    in_specs=[
        pl.BlockSpec((BN, HID), lambda i: (i, 0)),
        _full((HID, 32)),
        _full((1, 32)),
        _full((32, 16)),
        _full((1, 16)),
        _full((16, 10)),
        _full((1, 10)),
    ],
    out_specs=pl.BlockSpec((1, 10), lambda i: (0, 0)),
    out_shape=jax.ShapeDtypeStruct((1, 10), jnp.float32),
    scratch_shapes=[pltpu.VMEM((8, HID), jnp.float32)],
)


def kernel(nodes_feat, edge_index, edges_feat, nodes_num_norm_sqrt,
           edges_num_norm_sqrt, W_embed, b_embed, Wp, bp, Wa, ba,
           R0, rb0, R1, rb1, R2, rb2):
    src = edge_index[0]
    dst = edge_index[1]
    pad = EP - NE
    srcp = jnp.concatenate([src, jnp.zeros((pad,), jnp.int32)])
    dstp = jnp.concatenate([dst, jnp.full((pad,), DUMMY, jnp.int32)])
    srcd = jnp.stack([srcp * 8 + j for j in range(4)]).reshape(4, EP // BLK, BLK)
    dstp3 = dstp.reshape(EP // BLK, BLK)

    cnt2 = _sc_count(dstp.reshape(EPC, CH))

    h, q2 = _k0(nodes_feat, W_embed, b_embed.reshape(1, HID),
                Wp[0], bp[0].reshape(1, HID))
    for l in range(4):
        acc2 = _sc_aggregate(q2.reshape(8 * NN, 16), srcd, dstp3)
        if l < 3:
            h, q2 = _kb(h, acc2, cnt2, Wa[l], ba[l].reshape(1, HID),
                        Wp[l + 1], bp[l + 1].reshape(1, HID))
        else:
            h = _kb3(h, acc2, cnt2, Wa[l], ba[l].reshape(1, HID))

    return _kread(h, R0, rb0.reshape(1, 32), R1, rb1.reshape(1, 16),
                  R2, rb2.reshape(1, 10))


# speedup vs baseline: 10.2376x; 1.1136x over previous
import functools

import jax
import jax.numpy as jnp
from jax import lax
from jax.experimental import pallas as pl
from jax.experimental.pallas import tpu as pltpu
from jax.experimental.pallas import tpu_sc as plsc

NN = 50000
NE = 800000
HID = 64

CH = 128
KI = 14
BLK = KI * CH
NBLK = 28
NPAIR = NBLK // 2
TCH = 392
EPC = 16 * TCH
EP = EPC * CH
NPAD = 50048
DUMMY = 50000
TROW = 3128

_mesh = plsc.VectorSubcoreMesh(core_axis_name="c", subcore_axis_name="s")


def _zero_fill(ref, nrows, ncols):
    def row(j, _):
        for k in range(ncols // 16):
            ref[j, pl.ds(k * 16, 16)] = jnp.zeros((16,), jnp.float32)
        return 0
    lax.fori_loop(0, nrows, row, 0)


@functools.partial(
    pl.kernel,
    mesh=_mesh,
    out_type=jax.ShapeDtypeStruct((NPAD, 128), jnp.float32),
    scratch_types=[
        pltpu.VMEM((8, CH), jnp.int32),
        pltpu.VMEM((CH, 16), jnp.float32),
        pltpu.VMEM((CH, 16), jnp.float32),
        pltpu.VMEM_SHARED((NPAD, 16), jnp.float32),
        pltpu.SemaphoreType.DMA,
    ],
    compiler_params=pltpu.CompilerParams(use_tc_tiling_on_sc=False),
)
def _sc_count(dstp, out, didx, ones_v, zb, cnt, sem):
    cid = lax.axis_index("c")
    sid = lax.axis_index("s")

    def fill(j, _):
        ones_v[j, pl.ds(0, 16)] = jnp.ones((16,), jnp.float32)
        zb[j, pl.ds(0, 16)] = jnp.zeros((16,), jnp.float32)
        return 0
    lax.fori_loop(0, CH, fill, 0)

    zbase = sid * TROW
    for t in range(24):
        pltpu.sync_copy(zb, cnt.at[pl.ds(zbase + t * CH, CH)])
    pltpu.sync_copy(zb.at[pl.ds(0, 56)], cnt.at[pl.ds(zbase + 24 * CH, 56)])
    plsc.subcore_barrier()

    def body(g, _):
        cb = cid * (EPC // 2) + sid * 192 + g * 8
        pltpu.sync_copy(dstp.at[pl.ds(cb, 8)], didx)
        for j in range(8):
            pltpu.sync_copy(ones_v, cnt.at[didx.at[j]], add=True)
        return 0
    lax.fori_loop(0, 24, body, 0)

    @pl.when(sid < 8)
    def _():
        cb = cid * (EPC // 2) + 3072 + sid * 8
        pltpu.sync_copy(dstp.at[pl.ds(cb, 8)], didx)
        for j in range(8):
            pltpu.sync_copy(ones_v, cnt.at[didx.at[j]], add=True)

    plsc.subcore_barrier()

    wb = sid * TROW
    pltpu.sync_copy(cnt.at[pl.ds(wb, TROW)],
                    out.at[pl.ds(wb, TROW), pl.ds(16 * cid, 16)])


@functools.partial(
    pl.kernel,
    mesh=_mesh,
    out_type=jax.ShapeDtypeStruct((NPAD, 128), jnp.float32),
    scratch_types=[
        pltpu.VMEM((2, BLK), jnp.int32),
        pltpu.VMEM((2, BLK), jnp.int32),
        pltpu.VMEM((2, BLK, 16), jnp.float32),
        pltpu.VMEM((CH, 16), jnp.float32),
        pltpu.VMEM_SHARED((NPAD, 16), jnp.float32),
        pltpu.SemaphoreType.DMA,
        pltpu.SemaphoreType.DMA,
    ],
    compiler_params=pltpu.CompilerParams(use_tc_tiling_on_sc=False),
)
def _sc_aggregate(q4, srcd, dstp, out, sidx, didx, rows, zb, acc, semA, semB):
    cid = lax.axis_index("c")
    sid = lax.axis_index("s")

    _zero_fill(zb, CH, 16)
    for p in range(2):
        quarter = 2 * cid + p
        zbase = sid * TROW
        for t in range(24):
            pltpu.sync_copy(zb, acc.at[pl.ds(zbase + t * CH, CH)])
        pltpu.sync_copy(zb.at[pl.ds(0, 56)], acc.at[pl.ds(zbase + 24 * CH, 56)])
        plsc.subcore_barrier()

        base = sid * NBLK

        def load_and_fire(buf, cb, sem):
            pltpu.sync_copy(srcd.at[quarter, cb], sidx.at[buf])
            pltpu.sync_copy(dstp.at[cb], didx.at[buf])
            pltpu.async_copy(q4.at[sidx.at[buf]], rows.at[buf], sem)

        def drain(buf, sem):
            pltpu.make_async_copy(q4.at[sidx.at[buf]], rows.at[buf], sem).wait()

        def scatter(buf):
            pltpu.sync_copy(rows.at[buf], acc.at[didx.at[buf]], add=True)

        load_and_fire(0, base, semA)

        def body(b, _):
            load_and_fire(1, base + (2 * b + 1), semB)
            drain(0, semA)
            scatter(0)

            @pl.when(b < NPAIR - 1)
            def _():
                load_and_fire(0, base + (2 * b + 2), semA)

            drain(1, semB)
            scatter(1)
            return 0
        lax.fori_loop(0, NPAIR, body, 0)
        plsc.subcore_barrier()

        wb = sid * TROW
        pltpu.sync_copy(acc.at[pl.ds(wb, TROW)],
                        out.at[pl.ds(wb, TROW), pl.ds(16 * quarter, 16)])


BN = 2000
GRID = NN // BN


def _full(shape):
    return pl.BlockSpec(shape, lambda i: tuple(0 for _ in shape))


def _split_q(q, q_out):
    q_out[...] = jnp.concatenate([q, q], axis=1)


def _k0_body(nf, We, be, Wp0, bp0, h_out, q_out):
    h = jnp.dot(nf[...], We[...], preferred_element_type=jnp.float32) + be[...]
    q = jnp.maximum(
        jnp.dot(h, Wp0[...], preferred_element_type=jnp.float32) + bp0[...], 0.0)
    h_out[...] = h
    _split_q(q, q_out)


_k0 = pl.pallas_call(
    _k0_body,
    grid=(GRID,),
    in_specs=[
        pl.BlockSpec((BN, 128), lambda i: (i, 0)),
        _full((128, HID)),
        _full((1, HID)),
        _full((HID, HID)),
        _full((1, HID)),
    ],
    out_specs=[
        pl.BlockSpec((BN, HID), lambda i: (i, 0)),
        pl.BlockSpec((BN, 128), lambda i: (i, 0)),
    ],
    out_shape=[
        jax.ShapeDtypeStruct((NN, HID), jnp.float32),
        jax.ShapeDtypeStruct((NN, 128), jnp.float32),
    ],
)


def _node_apply(h, acc, cnt2, Wa, ba):
    cnt = cnt2[:, 0:1] + cnt2[:, 16:17]
    c = acc[:, 0:HID] / jnp.maximum(cnt, 1.0)
    bundle = (
        jnp.dot(h, Wa[:HID, :], preferred_element_type=jnp.float32)
        + jnp.dot(c, Wa[HID:, :], preferred_element_type=jnp.float32)
        + ba
    )
    nrm = jnp.sqrt(jnp.sum(bundle * bundle, axis=1, keepdims=True))
    bundle = bundle / jnp.maximum(nrm, 1e-12)
    return h + jnp.maximum(bundle, 0.0)


def _kb_body(h_in, acc_in, cnt_in, Wa, ba, Wpn, bpn, h_out, q_out):
    hn = _node_apply(h_in[...], acc_in[...], cnt_in[...], Wa[...], ba[...])
    q = jnp.maximum(
        jnp.dot(hn, Wpn[...], preferred_element_type=jnp.float32) + bpn[...], 0.0)
    h_out[...] = hn
    _split_q(q, q_out)


_kb = pl.pallas_call(
    _kb_body,
    grid=(GRID,),
    in_specs=[
        pl.BlockSpec((BN, HID), lambda i: (i, 0)),
        pl.BlockSpec((BN, 128), lambda i: (i, 0)),
        pl.BlockSpec((BN, 128), lambda i: (i, 0)),
        _full((2 * HID, HID)),
        _full((1, HID)),
        _full((HID, HID)),
        _full((1, HID)),
    ],
    out_specs=[
        pl.BlockSpec((BN, HID), lambda i: (i, 0)),
        pl.BlockSpec((BN, 128), lambda i: (i, 0)),
    ],
    out_shape=[
        jax.ShapeDtypeStruct((NN, HID), jnp.float32),
        jax.ShapeDtypeStruct((NN, 128), jnp.float32),
    ],
)


def _kb3_body(h_in, acc_in, cnt_in, Wa, ba, h_out):
    h_out[...] = _node_apply(h_in[...], acc_in[...], cnt_in[...], Wa[...], ba[...])


_kb3 = pl.pallas_call(
    _kb3_body,
    grid=(GRID,),
    in_specs=[
        pl.BlockSpec((BN, HID), lambda i: (i, 0)),
        pl.BlockSpec((BN, 128), lambda i: (i, 0)),
        pl.BlockSpec((BN, 128), lambda i: (i, 0)),
        _full((2 * HID, HID)),
        _full((1, HID)),
    ],
    out_specs=pl.BlockSpec((BN, HID), lambda i: (i, 0)),
    out_shape=jax.ShapeDtypeStruct((NN, HID), jnp.float32),
)


def _kread_body(h, R0, rb0, R1, rb1, R2, rb2, out, accv):
    i = pl.program_id(0)

    @pl.when(i == 0)
    def _():
        accv[...] = jnp.zeros_like(accv)

    accv[0:1, :] = accv[0:1, :] + jnp.sum(h[...], axis=0, keepdims=True)

    @pl.when(i == GRID - 1)
    def _():
        hg = accv[0:1, :] * (1.0 / NN)
        y = jnp.maximum(
            jnp.dot(hg, R0[...], preferred_element_type=jnp.float32) + rb0[...], 0.0)
        y = jnp.maximum(
            jnp.dot(y, R1[...], preferred_element_type=jnp.float32) + rb1[...], 0.0)
        out[...] = jnp.dot(y, R2[...], preferred_element_type=jnp.float32) + rb2[...]


_kread = pl.pallas_call(
    _kread_body,
    grid=(GRID,),
    in_specs=[
        pl.BlockSpec((BN, HID), lambda i: (i, 0)),
        _full((HID, 32)),
        _full((1, 32)),
        _full((32, 16)),
        _full((1, 16)),
        _full((16, 10)),
        _full((1, 10)),
    ],
    out_specs=pl.BlockSpec((1, 10), lambda i: (0, 0)),
    out_shape=jax.ShapeDtypeStruct((1, 10), jnp.float32),
    scratch_shapes=[pltpu.VMEM((8, HID), jnp.float32)],
)


def kernel(nodes_feat, edge_index, edges_feat, nodes_num_norm_sqrt,
           edges_num_norm_sqrt, W_embed, b_embed, Wp, bp, Wa, ba,
           R0, rb0, R1, rb1, R2, rb2):
    src = edge_index[0]
    dst = edge_index[1]
    pad = EP - NE
    srcp = jnp.concatenate([src, jnp.zeros((pad,), jnp.int32)])
    dstp = jnp.concatenate([dst, jnp.full((pad,), DUMMY, jnp.int32)])
    srcd = jnp.stack([srcp * 8 + j for j in range(4)]).reshape(4, EP // BLK, BLK)
    dstp3 = dstp.reshape(EP // BLK, BLK)

    cnt2 = _sc_count(dstp.reshape(EPC, CH))

    h, q2 = _k0(nodes_feat, W_embed, b_embed.reshape(1, HID),
                Wp[0], bp[0].reshape(1, HID))
    for l in range(4):
        acc2 = _sc_aggregate(q2.reshape(8 * NN, 16), srcd, dstp3)
        if l < 3:
            h, q2 = _kb(h, acc2, cnt2, Wa[l], ba[l].reshape(1, HID),
                        Wp[l + 1], bp[l + 1].reshape(1, HID))
        else:
            h = _kb3(h, acc2, cnt2, Wa[l], ba[l].reshape(1, HID))

    return _kread(h, R0, rb0.reshape(1, 32), R1, rb1.reshape(1, 16),
                  R2, rb2.reshape(1, 10))


# speedup vs baseline: 10.5212x; 1.0277x over previous
import functools

import jax
import jax.numpy as jnp
from jax import lax
from jax.experimental import pallas as pl
from jax.experimental.pallas import tpu as pltpu
from jax.experimental.pallas import tpu_sc as plsc

NN = 50000
NE = 800000
HID = 64

CH = 128
KI = 14
BLK = KI * CH
NBLK = 28
NPAIR = NBLK // 2
TCH = 392
EPC = 16 * TCH
EP = EPC * CH
NPAD = 50048
DUMMY = 50000
TROW = 3128

_mesh = plsc.VectorSubcoreMesh(core_axis_name="c", subcore_axis_name="s")


def _zero_fill(ref, nrows, ncols):
    def row(j, _):
        for k in range(ncols // 16):
            ref[j, pl.ds(k * 16, 16)] = jnp.zeros((16,), jnp.float32)
        return 0
    lax.fori_loop(0, nrows, row, 0)


@functools.partial(
    pl.kernel,
    mesh=_mesh,
    out_type=jax.ShapeDtypeStruct((NPAD, 128), jnp.float32),
    scratch_types=[
        pltpu.VMEM((8, CH), jnp.int32),
        pltpu.VMEM((CH, 16), jnp.float32),
        pltpu.VMEM((CH, 16), jnp.float32),
        pltpu.VMEM_SHARED((NPAD, 16), jnp.float32),
        pltpu.SemaphoreType.DMA,
    ],
    compiler_params=pltpu.CompilerParams(use_tc_tiling_on_sc=False),
)
def _sc_count(dstp, out, didx, ones_v, zb, cnt, sem):
    cid = lax.axis_index("c")
    sid = lax.axis_index("s")

    def fill(j, _):
        ones_v[j, pl.ds(0, 16)] = jnp.ones((16,), jnp.float32)
        zb[j, pl.ds(0, 16)] = jnp.zeros((16,), jnp.float32)
        return 0
    lax.fori_loop(0, CH, fill, 0)

    zbase = sid * TROW
    for t in range(24):
        pltpu.sync_copy(zb, cnt.at[pl.ds(zbase + t * CH, CH)])
    pltpu.sync_copy(zb.at[pl.ds(0, 56)], cnt.at[pl.ds(zbase + 24 * CH, 56)])
    plsc.subcore_barrier()

    def body(g, _):
        cb = cid * (EPC // 2) + sid * 192 + g * 8
        pltpu.sync_copy(dstp.at[pl.ds(cb, 8)], didx)
        for j in range(8):
            pltpu.sync_copy(ones_v, cnt.at[didx.at[j]], add=True)
        return 0
    lax.fori_loop(0, 24, body, 0)

    @pl.when(sid < 8)
    def _():
        cb = cid * (EPC // 2) + 3072 + sid * 8
        pltpu.sync_copy(dstp.at[pl.ds(cb, 8)], didx)
        for j in range(8):
            pltpu.sync_copy(ones_v, cnt.at[didx.at[j]], add=True)

    plsc.subcore_barrier()

    wb = sid * TROW
    pltpu.sync_copy(cnt.at[pl.ds(wb, TROW)],
                    out.at[pl.ds(wb, TROW), pl.ds(16 * cid, 16)])


@functools.partial(
    pl.kernel,
    mesh=_mesh,
    out_type=jax.ShapeDtypeStruct((NPAD, 128), jnp.float32),
    scratch_types=[
        pltpu.VMEM((2, BLK), jnp.int32),
        pltpu.VMEM((2, BLK), jnp.int32),
        pltpu.VMEM((2, BLK, 16), jnp.float32),
        pltpu.VMEM((CH, 16), jnp.float32),
        pltpu.VMEM_SHARED((NPAD, 16), jnp.float32),
        pltpu.SemaphoreType.DMA,
        pltpu.SemaphoreType.DMA,
    ],
    compiler_params=pltpu.CompilerParams(use_tc_tiling_on_sc=False),
)
def _sc_aggregate(q4, srcd, dstp, out, sidx, didx, rows, zb, acc, semA, semB):
    cid = lax.axis_index("c")
    sid = lax.axis_index("s")

    _zero_fill(zb, CH, 16)
    for p in range(2):
        quarter = 2 * cid + p
        zbase = sid * TROW
        for t in range(24):
            pltpu.sync_copy(zb, acc.at[pl.ds(zbase + t * CH, CH)])
        pltpu.sync_copy(zb.at[pl.ds(0, 56)], acc.at[pl.ds(zbase + 24 * CH, 56)])
        plsc.subcore_barrier()

        base = sid * NBLK

        def load_and_fire(buf, cb, sem):
            pltpu.sync_copy(srcd.at[quarter, cb], sidx.at[buf])
            pltpu.sync_copy(dstp.at[cb], didx.at[buf])
            pltpu.async_copy(q4.at[sidx.at[buf]], rows.at[buf], sem)

        def drain(buf, sem):
            pltpu.make_async_copy(q4.at[sidx.at[buf]], rows.at[buf], sem).wait()

        def scatter(buf):
            pltpu.sync_copy(rows.at[buf], acc.at[didx.at[buf]], add=True)

        load_and_fire(0, base, semA)

        def body(b, _):
            load_and_fire(1, base + (2 * b + 1), semB)
            drain(0, semA)
            scatter(0)

            @pl.when(b < NPAIR - 1)
            def _():
                load_and_fire(0, base + (2 * b + 2), semA)

            drain(1, semB)
            scatter(1)
            return 0
        lax.fori_loop(0, NPAIR, body, 0)
        plsc.subcore_barrier()

        wb = sid * TROW
        pltpu.sync_copy(acc.at[pl.ds(wb, TROW)],
                        out.at[pl.ds(wb, TROW), pl.ds(16 * quarter, 16)])


BN = 2000
GRID = NN // BN


def _full(shape):
    return pl.BlockSpec(shape, lambda i: tuple(0 for _ in shape))


def _split_q(q, q_out):
    q_out[...] = jnp.concatenate([q, q], axis=1)


def _k0_body(nf, We, be, Wp0, bp0, h_out, q_out):
    h = jnp.dot(nf[...], We[...], preferred_element_type=jnp.float32) + be[...]
    q = jnp.maximum(
        jnp.dot(h, Wp0[...], preferred_element_type=jnp.float32) + bp0[...], 0.0)
    h_out[...] = h
    _split_q(q, q_out)


_k0 = pl.pallas_call(
    _k0_body,
    grid=(GRID,),
    in_specs=[
        pl.BlockSpec((BN, 128), lambda i: (i, 0)),
        _full((128, HID)),
        _full((1, HID)),
        _full((HID, HID)),
        _full((1, HID)),
    ],
    out_specs=[
        pl.BlockSpec((BN, HID), lambda i: (i, 0)),
        pl.BlockSpec((BN, 128), lambda i: (i, 0)),
    ],
    out_shape=[
        jax.ShapeDtypeStruct((NN, HID), jnp.float32),
        jax.ShapeDtypeStruct((NN, 128), jnp.float32),
    ],
)


def _node_apply(h, inv, acc, Wa, ba):
    c = acc[:, 0:HID] * inv
    bundle = (
        jnp.dot(h, Wa[:HID, :], preferred_element_type=jnp.float32)
        + jnp.dot(c, Wa[HID:, :], preferred_element_type=jnp.float32)
        + ba
    )
    nrm = jnp.sqrt(jnp.sum(bundle * bundle, axis=1, keepdims=True))
    bundle = bundle / jnp.maximum(nrm, 1e-12)
    return h + jnp.maximum(bundle, 0.0)


def _kb0_body(h_in, acc_in, cnt_in, Wa, ba, Wpn, bpn, hx_out, q_out):
    cnt = cnt_in[:, 0:1] + cnt_in[:, 16:17]
    inv = 1.0 / jnp.maximum(cnt, 1.0)
    hn = _node_apply(h_in[...], inv, acc_in[...], Wa[...], ba[...])
    hx_out[...] = jnp.concatenate(
        [hn, jnp.broadcast_to(inv, (BN, HID))], axis=1)
    q = jnp.maximum(
        jnp.dot(hn, Wpn[...], preferred_element_type=jnp.float32) + bpn[...], 0.0)
    _split_q(q, q_out)


_kb0 = pl.pallas_call(
    _kb0_body,
    grid=(GRID,),
    in_specs=[
        pl.BlockSpec((BN, HID), lambda i: (i, 0)),
        pl.BlockSpec((BN, 128), lambda i: (i, 0)),
        pl.BlockSpec((BN, 128), lambda i: (i, 0)),
        _full((2 * HID, HID)),
        _full((1, HID)),
        _full((HID, HID)),
        _full((1, HID)),
    ],
    out_specs=[
        pl.BlockSpec((BN, 128), lambda i: (i, 0)),
        pl.BlockSpec((BN, 128), lambda i: (i, 0)),
    ],
    out_shape=[
        jax.ShapeDtypeStruct((NN, 128), jnp.float32),
        jax.ShapeDtypeStruct((NN, 128), jnp.float32),
    ],
)


def _kb_body(hx_in, acc_in, Wa, ba, Wpn, bpn, hx_out, q_out):
    hx = hx_in[...]
    hn = _node_apply(hx[:, :HID], hx[:, HID:HID + 1], acc_in[...],
                     Wa[...], ba[...])
    hx_out[...] = jnp.concatenate([hn, hx[:, HID:]], axis=1)
    q = jnp.maximum(
        jnp.dot(hn, Wpn[...], preferred_element_type=jnp.float32) + bpn[...], 0.0)
    _split_q(q, q_out)


_kb = pl.pallas_call(
    _kb_body,
    grid=(GRID,),
    in_specs=[
        pl.BlockSpec((BN, 128), lambda i: (i, 0)),
        pl.BlockSpec((BN, 128), lambda i: (i, 0)),
        _full((2 * HID, HID)),
        _full((1, HID)),
        _full((HID, HID)),
        _full((1, HID)),
    ],
    out_specs=[
        pl.BlockSpec((BN, 128), lambda i: (i, 0)),
        pl.BlockSpec((BN, 128), lambda i: (i, 0)),
    ],
    out_shape=[
        jax.ShapeDtypeStruct((NN, 128), jnp.float32),
        jax.ShapeDtypeStruct((NN, 128), jnp.float32),
    ],
)


def _kfin_body(hx_in, acc_in, Wa, ba, R0, rb0, R1, rb1, R2, rb2, out, accv):
    hx = hx_in[...]
    hn = _node_apply(hx[:, :HID], hx[:, HID:HID + 1], acc_in[...],
                     Wa[...], ba[...])
    i = pl.program_id(0)

    @pl.when(i == 0)
    def _():
        accv[...] = jnp.zeros_like(accv)

    accv[0:1, :] = accv[0:1, :] + jnp.sum(hn, axis=0, keepdims=True)

    @pl.when(i == GRID - 1)
    def _():
        hg = accv[0:1, :] * (1.0 / NN)
        y = jnp.maximum(
            jnp.dot(hg, R0[...], preferred_element_type=jnp.float32) + rb0[...], 0.0)
        y = jnp.maximum(
            jnp.dot(y, R1[...], preferred_element_type=jnp.float32) + rb1[...], 0.0)
        out[...] = jnp.dot(y, R2[...], preferred_element_type=jnp.float32) + rb2[...]


_kfin = pl.pallas_call(
    _kfin_body,
    grid=(GRID,),
    in_specs=[
        pl.BlockSpec((BN, 128), lambda i: (i, 0)),
        pl.BlockSpec((BN, 128), lambda i: (i, 0)),
        _full((2 * HID, HID)),
        _full((1, HID)),
        _full((HID, 32)),
        _full((1, 32)),
        _full((32, 16)),
        _full((1, 16)),
        _full((16, 10)),
        _full((1, 10)),
    ],
    out_specs=pl.BlockSpec((1, 10), lambda i: (0, 0)),
    out_shape=jax.ShapeDtypeStruct((1, 10), jnp.float32),
    scratch_shapes=[pltpu.VMEM((8, HID), jnp.float32)],
)


def kernel(nodes_feat, edge_index, edges_feat, nodes_num_norm_sqrt,
           edges_num_norm_sqrt, W_embed, b_embed, Wp, bp, Wa, ba,
           R0, rb0, R1, rb1, R2, rb2):
    src = edge_index[0]
    dst = edge_index[1]
    pad = EP - NE
    srcp = jnp.concatenate([src, jnp.zeros((pad,), jnp.int32)])
    dstp = jnp.concatenate([dst, jnp.full((pad,), DUMMY, jnp.int32)])
    srcd = jnp.stack([srcp * 8 + j for j in range(4)]).reshape(4, EP // BLK, BLK)
    dstp3 = dstp.reshape(EP // BLK, BLK)

    cnt2 = _sc_count(dstp.reshape(EPC, CH))

    h, q2 = _k0(nodes_feat, W_embed, b_embed.reshape(1, HID),
                Wp[0], bp[0].reshape(1, HID))

    acc2 = _sc_aggregate(q2.reshape(8 * NN, 16), srcd, dstp3)
    hx, q2 = _kb0(h, acc2, cnt2, Wa[0], ba[0].reshape(1, HID),
                  Wp[1], bp[1].reshape(1, HID))
    for l in (1, 2):
        acc2 = _sc_aggregate(q2.reshape(8 * NN, 16), srcd, dstp3)
        hx, q2 = _kb(hx, acc2, Wa[l], ba[l].reshape(1, HID),
                     Wp[l + 1], bp[l + 1].reshape(1, HID))
    acc2 = _sc_aggregate(q2.reshape(8 * NN, 16), srcd, dstp3)
    return _kfin(hx, acc2, Wa[3], ba[3].reshape(1, HID),
                 R0, rb0.reshape(1, 32), R1, rb1.reshape(1, 16),
                 R2, rb2.reshape(1, 10))


# speedup vs baseline: 11.7808x; 1.1197x over previous
import functools

import jax
import jax.numpy as jnp
from jax import lax
from jax.experimental import pallas as pl
from jax.experimental.pallas import tpu as pltpu
from jax.experimental.pallas import tpu_sc as plsc

NN = 50000
NE = 800000
HID = 64

CH = 128
KI = 14
BLK = KI * CH
NBLK = 28
NPAIR = NBLK // 2
TCH = 392
EPC = 16 * TCH
EP = EPC * CH
NPAD = 50048
NX = 50048
DUMMY = 50000
TROW = 3128

_mesh = plsc.VectorSubcoreMesh(core_axis_name="c", subcore_axis_name="s")


def _zero_fill(ref, nrows, ncols):
    def row(j, _):
        for k in range(ncols // 16):
            ref[j, pl.ds(k * 16, 16)] = jnp.zeros((16,), jnp.float32)
        return 0
    lax.fori_loop(0, nrows, row, 0)


@functools.partial(
    pl.kernel,
    mesh=_mesh,
    out_type=jax.ShapeDtypeStruct((NPAD, 128), jnp.float32),
    scratch_types=[
        pltpu.VMEM((8, CH), jnp.int32),
        pltpu.VMEM((CH, 16), jnp.float32),
        pltpu.VMEM((CH, 16), jnp.float32),
        pltpu.VMEM_SHARED((NPAD, 16), jnp.float32),
        pltpu.SemaphoreType.DMA,
    ],
    compiler_params=pltpu.CompilerParams(use_tc_tiling_on_sc=False),
)
def _sc_count(dstp, out, didx, ones_v, zb, cnt, sem):
    cid = lax.axis_index("c")
    sid = lax.axis_index("s")

    def fill(j, _):
        ones_v[j, pl.ds(0, 16)] = jnp.ones((16,), jnp.float32)
        zb[j, pl.ds(0, 16)] = jnp.zeros((16,), jnp.float32)
        return 0
    lax.fori_loop(0, CH, fill, 0)

    zbase = sid * TROW
    for t in range(24):
        pltpu.sync_copy(zb, cnt.at[pl.ds(zbase + t * CH, CH)])
    pltpu.sync_copy(zb.at[pl.ds(0, 56)], cnt.at[pl.ds(zbase + 24 * CH, 56)])
    plsc.subcore_barrier()

    def body(g, _):
        cb = cid * (EPC // 2) + sid * 192 + g * 8
        pltpu.sync_copy(dstp.at[pl.ds(cb, 8)], didx)
        for j in range(8):
            pltpu.sync_copy(ones_v, cnt.at[didx.at[j]], add=True)
        return 0
    lax.fori_loop(0, 24, body, 0)

    @pl.when(sid < 8)
    def _():
        cb = cid * (EPC // 2) + 3072 + sid * 8
        pltpu.sync_copy(dstp.at[pl.ds(cb, 8)], didx)
        for j in range(8):
            pltpu.sync_copy(ones_v, cnt.at[didx.at[j]], add=True)

    plsc.subcore_barrier()

    wb = sid * TROW
    pltpu.sync_copy(cnt.at[pl.ds(wb, TROW)],
                    out.at[pl.ds(wb, TROW), pl.ds(16 * cid, 16)])


@functools.partial(
    pl.kernel,
    mesh=_mesh,
    out_type=jax.ShapeDtypeStruct((NPAD, 128), jnp.bfloat16),
    scratch_types=[
        pltpu.VMEM((2, BLK), jnp.int32),
        pltpu.VMEM((2, BLK), jnp.int32),
        pltpu.VMEM((2, BLK, 32), jnp.bfloat16),
        pltpu.VMEM((CH, 32), jnp.bfloat16),
        pltpu.VMEM_SHARED((NPAD, 32), jnp.bfloat16),
        pltpu.SemaphoreType.DMA,
        pltpu.SemaphoreType.DMA,
    ],
    compiler_params=pltpu.CompilerParams(use_tc_tiling_on_sc=False),
)
def _sc_aggregate(q4, srcd, dstp, out, sidx, didx, rows, zb, acc, semA, semB):
    cid = lax.axis_index("c")
    sid = lax.axis_index("s")

    def zrow(j, _):
        zb[j, pl.ds(0, 32)] = jnp.zeros((32,), jnp.bfloat16)
        return 0
    lax.fori_loop(0, CH, zrow, 0)

    zbase = sid * TROW
    for t in range(24):
        pltpu.sync_copy(zb, acc.at[pl.ds(zbase + t * CH, CH)])
    pltpu.sync_copy(zb.at[pl.ds(0, 56)], acc.at[pl.ds(zbase + 24 * CH, 56)])
    plsc.subcore_barrier()

    base = sid * NBLK

    def load_and_fire(buf, cb, sem):
        pltpu.sync_copy(srcd.at[cid, cb], sidx.at[buf])
        pltpu.sync_copy(dstp.at[cb], didx.at[buf])
        pltpu.async_copy(q4.at[sidx.at[buf]], rows.at[buf], sem)

    def drain(buf, sem):
        pltpu.make_async_copy(q4.at[sidx.at[buf]], rows.at[buf], sem).wait()

    def scatter(buf):
        pltpu.sync_copy(rows.at[buf], acc.at[didx.at[buf]], add=True)

    load_and_fire(0, base, semA)

    def body(b, _):
        load_and_fire(1, base + (2 * b + 1), semB)
        drain(0, semA)
        scatter(0)

        @pl.when(b < NPAIR - 1)
        def _():
            load_and_fire(0, base + (2 * b + 2), semA)

        drain(1, semB)
        scatter(1)
        return 0
    lax.fori_loop(0, NPAIR, body, 0)
    plsc.subcore_barrier()

    wb = sid * TROW
    pltpu.sync_copy(acc.at[pl.ds(wb, TROW)],
                    out.at[pl.ds(wb, TROW), pl.ds(32 * cid, 32)])


BN = 2000
GRID = NN // BN


def _full(shape):
    return pl.BlockSpec(shape, lambda i: tuple(0 for _ in shape))


def _split_q(q, q_out):
    qb = q.astype(jnp.bfloat16)
    q_out[...] = jnp.concatenate([qb, qb], axis=1)


def _k0_body(nf, We, be, Wp0, bp0, h_out, q_out):
    h = jnp.dot(nf[...], We[...], preferred_element_type=jnp.float32) + be[...]
    q = jnp.maximum(
        jnp.dot(h, Wp0[...], preferred_element_type=jnp.float32) + bp0[...], 0.0)
    h_out[...] = h
    _split_q(q, q_out)


_k0 = pl.pallas_call(
    _k0_body,
    grid=(GRID,),
    in_specs=[
        pl.BlockSpec((BN, 128), lambda i: (i, 0)),
        _full((128, HID)),
        _full((1, HID)),
        _full((HID, HID)),
        _full((1, HID)),
    ],
    out_specs=[
        pl.BlockSpec((BN, HID), lambda i: (i, 0)),
        pl.BlockSpec((BN, 128), lambda i: (i, 0)),
    ],
    out_shape=[
        jax.ShapeDtypeStruct((NN, HID), jnp.float32),
        jax.ShapeDtypeStruct((NX, 128), jnp.bfloat16),
    ],
)


def _node_apply(h, inv, acc, Wa, ba):
    c = acc[:, 0:HID].astype(jnp.float32) * inv
    bundle = (
        jnp.dot(h, Wa[:HID, :], preferred_element_type=jnp.float32)
        + jnp.dot(c, Wa[HID:, :], preferred_element_type=jnp.float32)
        + ba
    )
    nrm = jnp.sqrt(jnp.sum(bundle * bundle, axis=1, keepdims=True))
    bundle = bundle / jnp.maximum(nrm, 1e-12)
    return h + jnp.maximum(bundle, 0.0)


def _kb0_body(h_in, acc_in, cnt_in, Wa, ba, Wpn, bpn, hx_out, q_out):
    cnt = cnt_in[:, 0:1] + cnt_in[:, 16:17]
    inv = 1.0 / jnp.maximum(cnt, 1.0)
    hn = _node_apply(h_in[...], inv, acc_in[...], Wa[...], ba[...])
    hx_out[...] = jnp.concatenate(
        [hn, jnp.broadcast_to(inv, (BN, HID))], axis=1)
    q = jnp.maximum(
        jnp.dot(hn, Wpn[...], preferred_element_type=jnp.float32) + bpn[...], 0.0)
    _split_q(q, q_out)


_kb0 = pl.pallas_call(
    _kb0_body,
    grid=(GRID,),
    in_specs=[
        pl.BlockSpec((BN, HID), lambda i: (i, 0)),
        pl.BlockSpec((BN, 128), lambda i: (i, 0)),
        pl.BlockSpec((BN, 128), lambda i: (i, 0)),
        _full((2 * HID, HID)),
        _full((1, HID)),
        _full((HID, HID)),
        _full((1, HID)),
    ],
    out_specs=[
        pl.BlockSpec((BN, 128), lambda i: (i, 0)),
        pl.BlockSpec((BN, 128), lambda i: (i, 0)),
    ],
    out_shape=[
        jax.ShapeDtypeStruct((NN, 128), jnp.float32),
        jax.ShapeDtypeStruct((NX, 128), jnp.bfloat16),
    ],
)


def _kb_body(hx_in, acc_in, Wa, ba, Wpn, bpn, hx_out, q_out):
    hx = hx_in[...]
    hn = _node_apply(hx[:, :HID], hx[:, HID:HID + 1], acc_in[...],
                     Wa[...], ba[...])
    hx_out[...] = jnp.concatenate([hn, hx[:, HID:]], axis=1)
    q = jnp.maximum(
        jnp.dot(hn, Wpn[...], preferred_element_type=jnp.float32) + bpn[...], 0.0)
    _split_q(q, q_out)


_kb = pl.pallas_call(
    _kb_body,
    grid=(GRID,),
    in_specs=[
        pl.BlockSpec((BN, 128), lambda i: (i, 0)),
        pl.BlockSpec((BN, 128), lambda i: (i, 0)),
        _full((2 * HID, HID)),
        _full((1, HID)),
        _full((HID, HID)),
        _full((1, HID)),
    ],
    out_specs=[
        pl.BlockSpec((BN, 128), lambda i: (i, 0)),
        pl.BlockSpec((BN, 128), lambda i: (i, 0)),
    ],
    out_shape=[
        jax.ShapeDtypeStruct((NN, 128), jnp.float32),
        jax.ShapeDtypeStruct((NX, 128), jnp.bfloat16),
    ],
)


def _kfin_body(hx_in, acc_in, Wa, ba, R0, rb0, R1, rb1, R2, rb2, out, accv):
    hx = hx_in[...]
    hn = _node_apply(hx[:, :HID], hx[:, HID:HID + 1], acc_in[...],
                     Wa[...], ba[...])
    i = pl.program_id(0)

    @pl.when(i == 0)
    def _():
        accv[...] = jnp.zeros_like(accv)

    accv[0:1, :] = accv[0:1, :] + jnp.sum(hn, axis=0, keepdims=True)

    @pl.when(i == GRID - 1)
    def _():
        hg = accv[0:1, :] * (1.0 / NN)
        y = jnp.maximum(
            jnp.dot(hg, R0[...], preferred_element_type=jnp.float32) + rb0[...], 0.0)
        y = jnp.maximum(
            jnp.dot(y, R1[...], preferred_element_type=jnp.float32) + rb1[...], 0.0)
        out[...] = jnp.dot(y, R2[...], preferred_element_type=jnp.float32) + rb2[...]


_kfin = pl.pallas_call(
    _kfin_body,
    grid=(GRID,),
    in_specs=[
        pl.BlockSpec((BN, 128), lambda i: (i, 0)),
        pl.BlockSpec((BN, 128), lambda i: (i, 0)),
        _full((2 * HID, HID)),
        _full((1, HID)),
        _full((HID, 32)),
        _full((1, 32)),
        _full((32, 16)),
        _full((1, 16)),
        _full((16, 10)),
        _full((1, 10)),
    ],
    out_specs=pl.BlockSpec((1, 10), lambda i: (0, 0)),
    out_shape=jax.ShapeDtypeStruct((1, 10), jnp.float32),
    scratch_shapes=[pltpu.VMEM((8, HID), jnp.float32)],
)


def kernel(nodes_feat, edge_index, edges_feat, nodes_num_norm_sqrt,
           edges_num_norm_sqrt, W_embed, b_embed, Wp, bp, Wa, ba,
           R0, rb0, R1, rb1, R2, rb2):
    src = edge_index[0]
    dst = edge_index[1]
    pad = EP - NE
    srcp = jnp.concatenate([src, jnp.zeros((pad,), jnp.int32)])
    dstp = jnp.concatenate([dst, jnp.full((pad,), DUMMY, jnp.int32)])
    srcd = jnp.stack([srcp * 4 + j for j in range(2)]).reshape(2, EP // BLK, BLK)
    dstp3 = dstp.reshape(EP // BLK, BLK)

    cnt2 = _sc_count(dstp.reshape(EPC, CH))

    h, q2 = _k0(nodes_feat, W_embed, b_embed.reshape(1, HID),
                Wp[0], bp[0].reshape(1, HID))

    acc2 = _sc_aggregate(q2.reshape(4 * NX, 32), srcd, dstp3)
    hx, q2 = _kb0(h, acc2, cnt2, Wa[0], ba[0].reshape(1, HID),
                  Wp[1], bp[1].reshape(1, HID))
    for l in (1, 2):
        acc2 = _sc_aggregate(q2.reshape(4 * NX, 32), srcd, dstp3)
        hx, q2 = _kb(hx, acc2, Wa[l], ba[l].reshape(1, HID),
                     Wp[l + 1], bp[l + 1].reshape(1, HID))
    acc2 = _sc_aggregate(q2.reshape(4 * NX, 32), srcd, dstp3)
    return _kfin(hx, acc2, Wa[3], ba[3].reshape(1, HID),
                 R0, rb0.reshape(1, 32), R1, rb1.reshape(1, 16),
                 R2, rb2.reshape(1, 10))


# speedup vs baseline: 11.8712x; 1.0077x over previous
import functools

import jax
import jax.numpy as jnp
from jax import lax
from jax.experimental import pallas as pl
from jax.experimental.pallas import tpu as pltpu
from jax.experimental.pallas import tpu_sc as plsc

NN = 50000
NE = 800000
HID = 64

CH = 128
KI = 14
BLK = KI * CH
NBLK = 28
NPAIR = NBLK // 2
TCH = 392
EPC = 16 * TCH
EP = EPC * CH
NPAD = 50048
NX = 50048
DUMMY = 50000
TROW = 3128

_mesh = plsc.VectorSubcoreMesh(core_axis_name="c", subcore_axis_name="s")


def _zero_fill(ref, nrows, ncols):
    def row(j, _):
        for k in range(ncols // 16):
            ref[j, pl.ds(k * 16, 16)] = jnp.zeros((16,), jnp.float32)
        return 0
    lax.fori_loop(0, nrows, row, 0)


@functools.partial(
    pl.kernel,
    mesh=_mesh,
    out_type=jax.ShapeDtypeStruct((NPAD, 128), jnp.float32),
    scratch_types=[
        pltpu.VMEM((2, BLK), jnp.int32),
        pltpu.VMEM((BLK, 16), jnp.float32),
        pltpu.VMEM((CH, 16), jnp.float32),
        pltpu.VMEM_SHARED((NPAD, 16), jnp.float32),
        pltpu.SemaphoreType.DMA,
    ],
    compiler_params=pltpu.CompilerParams(use_tc_tiling_on_sc=False),
)
def _sc_count(dstp, out, didx, ones_v, zb, cnt, sem):
    cid = lax.axis_index("c")
    sid = lax.axis_index("s")

    def fill(j, _):
        ones_v[j, pl.ds(0, 16)] = jnp.ones((16,), jnp.float32)
        return 0
    lax.fori_loop(0, BLK, fill, 0)
    _zero_fill(zb, CH, 16)

    zbase = sid * TROW
    for t in range(24):
        pltpu.sync_copy(zb, cnt.at[pl.ds(zbase + t * CH, CH)])
    pltpu.sync_copy(zb.at[pl.ds(0, 56)], cnt.at[pl.ds(zbase + 24 * CH, 56)])
    plsc.subcore_barrier()

    nb = EP // BLK // 32
    base = cid * (EP // BLK // 2) + sid * nb
    pltpu.sync_copy(dstp.at[base], didx.at[0])

    def body(g, _):
        @pl.when(g < nb - 1)
        def _():
            pltpu.sync_copy(dstp.at[base + g + 1], didx.at[(g + 1) % 2])
        pltpu.sync_copy(ones_v, cnt.at[didx.at[g % 2]], add=True)
        return 0
    lax.fori_loop(0, nb, body, 0)

    plsc.subcore_barrier()

    wb = sid * TROW
    pltpu.sync_copy(cnt.at[pl.ds(wb, TROW)],
                    out.at[pl.ds(wb, TROW), pl.ds(16 * cid, 16)])


@functools.partial(
    pl.kernel,
    mesh=_mesh,
    out_type=jax.ShapeDtypeStruct((NPAD, 128), jnp.bfloat16),
    scratch_types=[
        pltpu.VMEM((2, BLK), jnp.int32),
        pltpu.VMEM((2, BLK), jnp.int32),
        pltpu.VMEM((2, BLK, 32), jnp.bfloat16),
        pltpu.VMEM((CH, 32), jnp.bfloat16),
        pltpu.VMEM_SHARED((NPAD, 32), jnp.bfloat16),
        pltpu.SemaphoreType.DMA,
        pltpu.SemaphoreType.DMA,
    ],
    compiler_params=pltpu.CompilerParams(use_tc_tiling_on_sc=False),
)
def _sc_aggregate(q4, srcd, dstp, out, sidx, didx, rows, zb, acc, semA, semB):
    cid = lax.axis_index("c")
    sid = lax.axis_index("s")

    def zrow(j, _):
        zb[j, pl.ds(0, 32)] = jnp.zeros((32,), jnp.bfloat16)
        return 0
    lax.fori_loop(0, CH, zrow, 0)

    zbase = sid * TROW
    for t in range(24):
        pltpu.sync_copy(zb, acc.at[pl.ds(zbase + t * CH, CH)])
    pltpu.sync_copy(zb.at[pl.ds(0, 56)], acc.at[pl.ds(zbase + 24 * CH, 56)])
    plsc.subcore_barrier()

    base = sid * NBLK

    def load_and_fire(buf, cb, sem):
        pltpu.sync_copy(srcd.at[cid, cb], sidx.at[buf])
        pltpu.sync_copy(dstp.at[cb], didx.at[buf])
        pltpu.async_copy(q4.at[sidx.at[buf]], rows.at[buf], sem)

    def drain(buf, sem):
        pltpu.make_async_copy(q4.at[sidx.at[buf]], rows.at[buf], sem).wait()

    def scatter(buf):
        pltpu.sync_copy(rows.at[buf], acc.at[didx.at[buf]], add=True)

    load_and_fire(0, base, semA)

    def body(b, _):
        load_and_fire(1, base + (2 * b + 1), semB)
        drain(0, semA)
        scatter(0)

        @pl.when(b < NPAIR - 1)
        def _():
            load_and_fire(0, base + (2 * b + 2), semA)

        drain(1, semB)
        scatter(1)
        return 0
    lax.fori_loop(0, NPAIR, body, 0)
    plsc.subcore_barrier()

    wb = sid * TROW
    pltpu.sync_copy(acc.at[pl.ds(wb, TROW)],
                    out.at[pl.ds(wb, TROW), pl.ds(32 * cid, 32)])


BN = 2000
GRID = NN // BN


def _full(shape):
    return pl.BlockSpec(shape, lambda i: tuple(0 for _ in shape))


def _split_q(q, q_out):
    qb = q.astype(jnp.bfloat16)
    q_out[...] = jnp.concatenate([qb, qb], axis=1)


def _k0_body(nf, We, be, Wp0, bp0, h_out, q_out):
    h = jnp.dot(nf[...], We[...], preferred_element_type=jnp.float32) + be[...]
    q = jnp.maximum(
        jnp.dot(h, Wp0[...], preferred_element_type=jnp.float32) + bp0[...], 0.0)
    h_out[...] = h
    _split_q(q, q_out)


_k0 = pl.pallas_call(
    _k0_body,
    grid=(GRID,),
    in_specs=[
        pl.BlockSpec((BN, 128), lambda i: (i, 0)),
        _full((128, HID)),
        _full((1, HID)),
        _full((HID, HID)),
        _full((1, HID)),
    ],
    out_specs=[
        pl.BlockSpec((BN, HID), lambda i: (i, 0)),
        pl.BlockSpec((BN, 128), lambda i: (i, 0)),
    ],
    out_shape=[
        jax.ShapeDtypeStruct((NN, HID), jnp.float32),
        jax.ShapeDtypeStruct((NX, 128), jnp.bfloat16),
    ],
)


def _node_apply(h, inv, acc, Wa, ba):
    c = acc[:, 0:HID].astype(jnp.float32) * inv
    bundle = (
        jnp.dot(h, Wa[:HID, :], preferred_element_type=jnp.float32)
        + jnp.dot(c, Wa[HID:, :], preferred_element_type=jnp.float32)
        + ba
    )
    nrm = jnp.sqrt(jnp.sum(bundle * bundle, axis=1, keepdims=True))
    bundle = bundle / jnp.maximum(nrm, 1e-12)
    return h + jnp.maximum(bundle, 0.0)


def _kb0_body(h_in, acc_in, cnt_in, Wa, ba, Wpn, bpn, hx_out, q_out):
    cnt = cnt_in[:, 0:1] + cnt_in[:, 16:17]
    inv = 1.0 / jnp.maximum(cnt, 1.0)
    hn = _node_apply(h_in[...], inv, acc_in[...], Wa[...], ba[...])
    hx_out[...] = jnp.concatenate(
        [hn, jnp.broadcast_to(inv, (BN, HID))], axis=1)
    q = jnp.maximum(
        jnp.dot(hn, Wpn[...], preferred_element_type=jnp.float32) + bpn[...], 0.0)
    _split_q(q, q_out)


_kb0 = pl.pallas_call(
    _kb0_body,
    grid=(GRID,),
    in_specs=[
        pl.BlockSpec((BN, HID), lambda i: (i, 0)),
        pl.BlockSpec((BN, 128), lambda i: (i, 0)),
        pl.BlockSpec((BN, 128), lambda i: (i, 0)),
        _full((2 * HID, HID)),
        _full((1, HID)),
        _full((HID, HID)),
        _full((1, HID)),
    ],
    out_specs=[
        pl.BlockSpec((BN, 128), lambda i: (i, 0)),
        pl.BlockSpec((BN, 128), lambda i: (i, 0)),
    ],
    out_shape=[
        jax.ShapeDtypeStruct((NN, 128), jnp.float32),
        jax.ShapeDtypeStruct((NX, 128), jnp.bfloat16),
    ],
)


def _kb_body(hx_in, acc_in, Wa, ba, Wpn, bpn, hx_out, q_out):
    hx = hx_in[...]
    hn = _node_apply(hx[:, :HID], hx[:, HID:HID + 1], acc_in[...],
                     Wa[...], ba[...])
    hx_out[...] = jnp.concatenate([hn, hx[:, HID:]], axis=1)
    q = jnp.maximum(
        jnp.dot(hn, Wpn[...], preferred_element_type=jnp.float32) + bpn[...], 0.0)
    _split_q(q, q_out)


_kb = pl.pallas_call(
    _kb_body,
    grid=(GRID,),
    in_specs=[
        pl.BlockSpec((BN, 128), lambda i: (i, 0)),
        pl.BlockSpec((BN, 128), lambda i: (i, 0)),
        _full((2 * HID, HID)),
        _full((1, HID)),
        _full((HID, HID)),
        _full((1, HID)),
    ],
    out_specs=[
        pl.BlockSpec((BN, 128), lambda i: (i, 0)),
        pl.BlockSpec((BN, 128), lambda i: (i, 0)),
    ],
    out_shape=[
        jax.ShapeDtypeStruct((NN, 128), jnp.float32),
        jax.ShapeDtypeStruct((NX, 128), jnp.bfloat16),
    ],
)


def _kfin_body(hx_in, acc_in, Wa, ba, R0, rb0, R1, rb1, R2, rb2, out, accv):
    hx = hx_in[...]
    hn = _node_apply(hx[:, :HID], hx[:, HID:HID + 1], acc_in[...],
                     Wa[...], ba[...])
    i = pl.program_id(0)

    @pl.when(i == 0)
    def _():
        accv[...] = jnp.zeros_like(accv)

    accv[0:1, :] = accv[0:1, :] + jnp.sum(hn, axis=0, keepdims=True)

    @pl.when(i == GRID - 1)
    def _():
        hg = accv[0:1, :] * (1.0 / NN)
        y = jnp.maximum(
            jnp.dot(hg, R0[...], preferred_element_type=jnp.float32) + rb0[...], 0.0)
        y = jnp.maximum(
            jnp.dot(y, R1[...], preferred_element_type=jnp.float32) + rb1[...], 0.0)
        out[...] = jnp.dot(y, R2[...], preferred_element_type=jnp.float32) + rb2[...]


_kfin = pl.pallas_call(
    _kfin_body,
    grid=(GRID,),
    in_specs=[
        pl.BlockSpec((BN, 128), lambda i: (i, 0)),
        pl.BlockSpec((BN, 128), lambda i: (i, 0)),
        _full((2 * HID, HID)),
        _full((1, HID)),
        _full((HID, 32)),
        _full((1, 32)),
        _full((32, 16)),
        _full((1, 16)),
        _full((16, 10)),
        _full((1, 10)),
    ],
    out_specs=pl.BlockSpec((1, 10), lambda i: (0, 0)),
    out_shape=jax.ShapeDtypeStruct((1, 10), jnp.float32),
    scratch_shapes=[pltpu.VMEM((8, HID), jnp.float32)],
)


def kernel(nodes_feat, edge_index, edges_feat, nodes_num_norm_sqrt,
           edges_num_norm_sqrt, W_embed, b_embed, Wp, bp, Wa, ba,
           R0, rb0, R1, rb1, R2, rb2):
    src = edge_index[0]
    dst = edge_index[1]
    pad = EP - NE
    srcp = jnp.concatenate([src, jnp.zeros((pad,), jnp.int32)])
    dstp = jnp.concatenate([dst, jnp.full((pad,), DUMMY, jnp.int32)])
    srcd = jnp.stack([srcp * 4 + j for j in range(2)]).reshape(2, EP // BLK, BLK)
    dstp3 = dstp.reshape(EP // BLK, BLK)

    cnt2 = _sc_count(dstp3)

    h, q2 = _k0(nodes_feat, W_embed, b_embed.reshape(1, HID),
                Wp[0], bp[0].reshape(1, HID))

    acc2 = _sc_aggregate(q2.reshape(4 * NX, 32), srcd, dstp3)
    hx, q2 = _kb0(h, acc2, cnt2, Wa[0], ba[0].reshape(1, HID),
                  Wp[1], bp[1].reshape(1, HID))
    for l in (1, 2):
        acc2 = _sc_aggregate(q2.reshape(4 * NX, 32), srcd, dstp3)
        hx, q2 = _kb(hx, acc2, Wa[l], ba[l].reshape(1, HID),
                     Wp[l + 1], bp[l + 1].reshape(1, HID))
    acc2 = _sc_aggregate(q2.reshape(4 * NX, 32), srcd, dstp3)
    return _kfin(hx, acc2, Wa[3], ba[3].reshape(1, HID),
                 R0, rb0.reshape(1, 32), R1, rb1.reshape(1, 16),
                 R2, rb2.reshape(1, 10))
